# baseline (speedup 1.0000x reference)
import functools

import jax
import jax.numpy as jnp
from jax import lax
from jax.experimental import pallas as pl
from jax.experimental.pallas import tpu as pltpu
from jax.experimental.pallas import tpu_sc as plsc

N = 10000
E = 320000
H = 64
D_EDGE = 16

NT = 10240
TW = 128
NW = 32
EPAD = 327680
EPT = EPAD // NW
GC = 128
NCH = EPT // GC
SB = 256
NSB = EPT // SB
EB = 2048
NEB = EPAD // EB

F32 = jnp.float32

@functools.cache
def _sc_mesh():
    return plsc.VectorSubcoreMesh(core_axis_name="c", subcore_axis_name="s",
                                  num_cores=2, num_subcores=16)


def _silu(x):
    return x * jax.nn.sigmoid(x)


def _gather_pass(table, idx, out, wid, buf0, buf1, semg0, semg1, semw0,
                 semw1):
    bufs = (buf0, buf1)
    semg = (semg0, semg1)
    semw = (semw0, semw1)
    base = wid * EPT

    def issue(c, b):
        for j in range(SB // GC):
            pltpu.async_copy(table.at[idx.at[c * (SB // GC) + j]],
                             bufs[b].at[pl.ds(j * GC, GC)], semg[b])

    def wait_gather(b):
        for j in range(SB // GC):
            pltpu.make_async_copy(table.at[idx.at[0]],
                                  bufs[b].at[pl.ds(j * GC, GC)],
                                  semg[b]).wait()

    def wait_write(b):
        pltpu.make_async_copy(bufs[b], out.at[pl.ds(base, SB)],
                              semw[b]).wait()

    issue(0, 0)

    @pl.loop(0, NSB, step=2)
    def _(i):
        for b in range(2):
            c = i + b

            @pl.when(c + 1 < NSB)
            def _():
                @pl.when(c >= 1)
                def _():
                    wait_write(1 - b)

                issue(c + 1, 1 - b)

            wait_gather(b)
            pltpu.async_copy(bufs[b], out.at[pl.ds(base + c * SB, SB)],
                             semw[b])

    wait_write(0)
    wait_write(1)


def _gather_body(row2, col2, tsrc, tdst, gsrc, gdst, idxr, idxc, buf0, buf1,
                 semg0, semg1, semw0, semw1):
    cid = lax.axis_index("c")
    sid = lax.axis_index("s")
    wid = sid * 2 + cid
    pltpu.sync_copy(row2.at[pl.ds(wid * NCH, NCH)], idxr)
    pltpu.sync_copy(col2.at[pl.ds(wid * NCH, NCH)], idxc)
    _gather_pass(tsrc, idxr, gsrc, wid, buf0, buf1, semg0, semg1, semw0,
                 semw1)
    _gather_pass(tdst, idxc, gdst, wid, buf0, buf1, semg0, semg1, semw0,
                 semw1)


@functools.cache
def _gather_call():
    return pl.kernel(
        _gather_body,
        out_type=(jax.ShapeDtypeStruct((EPAD, TW), F32),
                  jax.ShapeDtypeStruct((EPAD, TW), F32)),
        mesh=_sc_mesh(),
        scratch_types=[
            pltpu.VMEM((NCH, GC), jnp.int32),
            pltpu.VMEM((NCH, GC), jnp.int32),
            pltpu.VMEM((SB, TW), F32),
            pltpu.VMEM((SB, TW), F32),
            pltpu.SemaphoreType.DMA,
            pltpu.SemaphoreType.DMA,
            pltpu.SemaphoreType.DMA,
            pltpu.SemaphoreType.DMA,
        ],
    )


def _gather_pallas(row2, col2, tsrc, tdst):
    return _gather_call()(row2, col2, tsrc, tdst)


def _scatter_body(row2, pay, zeros, part, idx2, buf0, buf1, acc, seml0,
                  seml1):
    cid = lax.axis_index("c")
    sid = lax.axis_index("s")
    wid = sid * 2 + cid
    bufs = (buf0, buf1)
    seml = (seml0, seml1)

    @pl.when(sid == 0)
    def _():
        pltpu.sync_copy(zeros, acc)

    pltpu.sync_copy(row2.at[pl.ds(wid * NCH, NCH)], idx2)
    plsc.subcore_barrier()

    def issue(c, b):
        pltpu.async_copy(pay.at[pl.ds(wid * EPT + c * GC, GC)], bufs[b],
                         seml[b])

    def wait_load(b):
        pltpu.make_async_copy(pay.at[pl.ds(0, GC)], bufs[b], seml[b]).wait()

    issue(0, 0)

    @pl.loop(0, NCH, step=2)
    def _(i):
        for b in range(2):
            c = i + b
            wait_load(b)

            @pl.when(c + 1 < NCH)
            def _():
                issue(c + 1, 1 - b)

            pltpu.sync_copy(bufs[b], acc.at[idx2.at[c]], add=True)

    plsc.subcore_barrier()

    @pl.when(sid == 0)
    def _():
        pltpu.sync_copy(acc, part.at[cid])


@functools.cache
def _scatter_call():
    return pl.kernel(
        _scatter_body,
        out_type=jax.ShapeDtypeStruct((2, NT, TW), F32),
        mesh=_sc_mesh(),
        scratch_types=[
            pltpu.VMEM((NCH, GC), jnp.int32),
            pltpu.VMEM((GC, TW), F32),
            pltpu.VMEM((GC, TW), F32),
            pltpu.VMEM_SHARED((NT, TW), F32),
            pltpu.SemaphoreType.DMA,
            pltpu.SemaphoreType.DMA,
        ],
    )


def _scatter_pallas(row2, payload, zeros):
    return _scatter_call()(row2, payload, zeros)


def _node0_body(his_ref, wemb_ref, bemb_ref, vel_ref, wsrc_ref, wdst_ref,
                wv1_ref, bv1_ref, wv2t_ref, bv2_ref,
                h_ref, hs_ref, hd_ref, vmod_ref, nvel_ref):
    h = jnp.dot(his_ref[...], wemb_ref[...], preferred_element_type=F32)
    h = h + bemb_ref[...]
    h_ref[...] = h
    hs_ref[...] = jnp.dot(h, wsrc_ref[...], preferred_element_type=F32)
    hd_ref[...] = jnp.dot(h, wdst_ref[...], preferred_element_type=F32)
    q = _silu(jnp.dot(h, wv1_ref[...], preferred_element_type=F32)
              + bv1_ref[...])
    vmod_ref[...] = (jnp.sum(q * wv2t_ref[...], axis=1, keepdims=True)
                     + bv2_ref[...])
    v = vel_ref[...]
    nrm = jnp.sqrt(jnp.sum(v * v, axis=1, keepdims=True)) + 1e-8
    nvel_ref[...] = v / nrm


NB = 1000
NNB = N // NB


def _node0_pallas(his, vel, wemb, bemb, wsrc, wdst, wv1, bv1, wv2t, bv2):
    blk = lambda c: pl.BlockSpec((NB, c), lambda i: (i, 0))
    rep = lambda r, c: pl.BlockSpec((r, c), lambda i: (0, 0))
    return pl.pallas_call(
        _node0_body,
        grid=(NNB,),
        in_specs=[
            blk(128), rep(128, H), rep(1, H), blk(3), rep(H, H), rep(H, H),
            rep(H, H), rep(1, H), rep(1, H), rep(1, 1),
        ],
        out_specs=(blk(H), blk(H), blk(H), blk(1), blk(3)),
        out_shape=(
            jax.ShapeDtypeStruct((N, H), F32),
            jax.ShapeDtypeStruct((N, H), F32),
            jax.ShapeDtypeStruct((N, H), F32),
            jax.ShapeDtypeStruct((N, 1), F32),
            jax.ShapeDtypeStruct((N, 3), F32),
        ),
    )(his, wemb, bemb, vel, wsrc, wdst, wv1, bv1, wv2t, bv2)


def _eaproj_body(ea_ref, wea_ref, be1_ref, out_ref):
    out_ref[...] = (jnp.dot(ea_ref[...], wea_ref[...],
                            preferred_element_type=F32) + be1_ref[...])


def _eaproj_pallas(ea_pad, wea, be1):
    return pl.pallas_call(
        _eaproj_body,
        grid=(NEB,),
        in_specs=[
            pl.BlockSpec((EB, D_EDGE), lambda i: (i, 0)),
            pl.BlockSpec((D_EDGE, H), lambda i: (0, 0)),
            pl.BlockSpec((1, H), lambda i: (0, 0)),
        ],
        out_specs=pl.BlockSpec((EB, H), lambda i: (i, 0)),
        out_shape=jax.ShapeDtypeStruct((EPAD, H), F32),
    )(ea_pad, wea, be1)


def _edge_body(gs_ref, gd_ref, eab_ref, radw_ref, we2_ref, be2_ref,
               wc1_ref, bc1_ref, wc2t_ref, out_ref):
    gs = gs_ref[...]
    gd = gd_ref[...]
    cd = gs[:, H:H + 16] - gd[:, H:H + 16]
    radial = jnp.sum(cd * cd, axis=1, keepdims=True)
    pre1 = gs[:, 0:H] + gd[:, 0:H] + eab_ref[...] + radial * radw_ref[...]
    m1 = _silu(pre1)
    m = _silu(jnp.dot(m1, we2_ref[...], preferred_element_type=F32)
              + be2_ref[...])
    q = _silu(jnp.dot(m, wc1_ref[...], preferred_element_type=F32)
              + bc1_ref[...])
    cm = jnp.sum(q * wc2t_ref[...], axis=1, keepdims=True)
    t16 = cd * cm
    lane = lax.broadcasted_iota(jnp.int32, t16.shape, 1)
    t16 = jnp.where(lane == 3, 1.0, t16)
    out_ref[...] = jnp.concatenate(
        [t16, m, jnp.zeros((t16.shape[0], TW - 16 - H), F32)], axis=1)


def _edge_pallas(gsrc, gdst, eab, radw, we2, be2, wc1, bc1, wc2t):
    full = lambda s: pl.BlockSpec(s, lambda i: tuple(0 for _ in s))
    return pl.pallas_call(
        _edge_body,
        grid=(NEB,),
        in_specs=[
            pl.BlockSpec((EB, TW), lambda i: (i, 0)),
            pl.BlockSpec((EB, TW), lambda i: (i, 0)),
            pl.BlockSpec((EB, H), lambda i: (i, 0)),
            full((1, H)),
            full((H, H)),
            full((1, H)),
            full((H, H)),
            full((1, H)),
            full((1, H)),
        ],
        out_specs=pl.BlockSpec((EB, TW), lambda i: (i, 0)),
        out_shape=jax.ShapeDtypeStruct((EPAD, TW), F32),
    )(gsrc, gdst, eab, radw, we2, be2, wc1, bc1, wc2t)


def _node_body(p_ref, h_ref, x_ref, v_ref, nvel_ref, vmod_ref,
               wnh_ref, wnm_ref, bn1_ref, wn2_ref, bn2_ref,
               wsrc_ref, wdst_ref, wv1_ref, bv1_ref, wv2t_ref, bv2_ref,
               hn_ref, xn_ref, vn_ref, hs_ref, hd_ref, vmodn_ref):
    ps = p_ref[0] + p_ref[1]
    cnt = jnp.maximum(ps[:, 3:4], 1.0)
    agg = ps[:, 0:3] / cnt
    v = v_ref[...] + agg + vmod_ref[...] * nvel_ref[...]
    vn_ref[...] = v
    xn_ref[...] = x_ref[...] + v
    magg = ps[:, 16:80]
    h = h_ref[...]
    hmid = _silu(jnp.dot(h, wnh_ref[...], preferred_element_type=F32)
                 + jnp.dot(magg, wnm_ref[...], preferred_element_type=F32)
                 + bn1_ref[...])
    hn = h + jnp.dot(hmid, wn2_ref[...], preferred_element_type=F32) \
        + bn2_ref[...]
    hn_ref[...] = hn
    hs_ref[...] = jnp.dot(hn, wsrc_ref[...], preferred_element_type=F32)
    hd_ref[...] = jnp.dot(hn, wdst_ref[...], preferred_element_type=F32)
    q = _silu(jnp.dot(hn, wv1_ref[...], preferred_element_type=F32)
              + bv1_ref[...])
    vmodn_ref[...] = (jnp.sum(q * wv2t_ref[...], axis=1, keepdims=True)
                      + bv2_ref[...])


def _node_pallas(partials, h, x, v, nvel, vmod, wnh, wnm, bn1, wn2, bn2,
                 wsrc, wdst, wv1, bv1, wv2t, bv2):
    blk = lambda c: pl.BlockSpec((NB, c), lambda i: (i, 0))
    rep = lambda r, c: pl.BlockSpec((r, c), lambda i: (0, 0))
    return pl.pallas_call(
        _node_body,
        grid=(NNB,),
        in_specs=[
            pl.BlockSpec((2, NB, TW), lambda i: (0, i, 0)),
            blk(H), blk(3), blk(3), blk(3), blk(1),
            rep(H, H), rep(H, H), rep(1, H), rep(H, H), rep(1, H),
            rep(H, H), rep(H, H), rep(H, H), rep(1, H), rep(1, H), rep(1, 1),
        ],
        out_specs=(blk(H), blk(3), blk(3), blk(H), blk(H), blk(1)),
        out_shape=(
            jax.ShapeDtypeStruct((N, H), F32),
            jax.ShapeDtypeStruct((N, 3), F32),
            jax.ShapeDtypeStruct((N, 3), F32),
            jax.ShapeDtypeStruct((N, H), F32),
            jax.ShapeDtypeStruct((N, H), F32),
            jax.ShapeDtypeStruct((N, 1), F32),
        ),
    )(partials, h, x, v, nvel, vmod, wnh, wnm, bn1, wn2, bn2,
      wsrc, wdst, wv1, bv1, wv2t, bv2)


def _table(hproj, x):
    t = jnp.concatenate([hproj, x, jnp.zeros((N, TW - H - 3), F32)], axis=1)
    return jnp.pad(t, ((0, NT - N), (0, 0)))


def kernel(his, loc, edges, vel, edge_attr, W_emb, b_emb, We1, be1, We2, be2,
           Wn1, bn1, Wn2, bn2, Wc1, bc1, Wc2, Wv1, bv1, Wv2, bv2):
    row = edges[0]
    col = edges[1]
    pad = jnp.full((EPAD - E,), N, jnp.int32)
    row2 = jnp.concatenate([row, pad]).reshape(EPAD // GC, GC)
    col2 = jnp.concatenate([col, pad]).reshape(EPAD // GC, GC)
    ea_pad = jnp.pad(edge_attr, ((0, EPAD - E), (0, 0)))
    zeros = jnp.zeros((NT, TW), F32)

    wsrc = We1[0:H]
    wdst = We1[H:2 * H]
    radw = We1[2 * H:2 * H + 1]
    wea = We1[2 * H + 1:]
    wnh = Wn1[0:H]
    wnm = Wn1[H:2 * H]
    b_emb2 = b_emb.reshape(1, H)
    be1_2 = be1.reshape(1, H)
    be2_2 = be2.reshape(1, H)
    bn1_2 = bn1.reshape(1, H)
    bn2_2 = bn2.reshape(1, H)
    bc1_2 = bc1.reshape(1, H)
    bv1_2 = bv1.reshape(1, H)
    wc2t = Wc2.reshape(1, H)
    wv2t = Wv2.reshape(1, H)
    bv2_2 = bv2.reshape(1, 1)

    h, hs, hd, vmod, nvel = _node0_pallas(
        his, vel, W_emb, b_emb2, wsrc, wdst, Wv1, bv1_2, wv2t, bv2_2)
    eab = _eaproj_pallas(ea_pad, wea, be1_2)

    x = loc
    v = vel
    for _ in range(4):
        tsrc = _table(hs, x)
        tdst = _table(hd, x)
        gsrc, gdst = _gather_pallas(row2, col2, tsrc, tdst)
        payload = _edge_pallas(gsrc, gdst, eab, radw, We2, be2_2,
                               Wc1, bc1_2, wc2t)
        partials = _scatter_pallas(row2, payload, zeros)
        h, x, v, hs, hd, vmod = _node_pallas(
            partials, h, x, v, nvel, vmod, wnh, wnm, bn1_2, Wn2, bn2_2,
            wsrc, wdst, Wv1, bv1_2, wv2t, bv2_2)
    return x, h, v


# speedup vs baseline: 1.9848x; 1.9848x over previous
import functools

import jax
import jax.numpy as jnp
from jax import lax
from jax.experimental import pallas as pl
from jax.experimental.pallas import tpu as pltpu
from jax.experimental.pallas import tpu_sc as plsc

N = 10000
E = 320000
H = 64
D_EDGE = 16

NT = 10240
TW = 128
NW = 32
EPAD = 327680
EPT = EPAD // NW
GC = 128
NCH = EPT // GC
SB = 256
NSB = EPT // SB
EB = 2048
NEB = EPAD // EB

F32 = jnp.float32

@functools.cache
def _sc_mesh():
    return plsc.VectorSubcoreMesh(core_axis_name="c", subcore_axis_name="s",
                                  num_cores=2, num_subcores=16)


def _silu(x):
    return x * jax.nn.sigmoid(x)


EPT2 = EPAD // 16
IST = 80
NST = EPT2 // (IST * GC)
NTS = NT // 16


def _gather_pipe(idx2, out, sid, idx, buf0, buf1, tab, semw0, semw1):
    bufs = (buf0, buf1)
    semw = (semw0, semw1)

    def wait_write(b):
        pltpu.make_async_copy(bufs[b], out.at[pl.ds(0, GC)], semw[b]).wait()

    for stage in range(NST):
        pltpu.sync_copy(idx2.at[pl.ds(sid * (EPT2 // GC) + stage * IST, IST)],
                        idx)

        @pl.loop(0, IST, step=2)
        def _(k):
            for b in range(2):
                if stage == 0:
                    @pl.when(k + b >= 2)
                    def _():
                        wait_write(b)
                else:
                    wait_write(b)
                pltpu.sync_copy(tab.at[idx.at[k + b]], bufs[b])
                base = sid * EPT2 + stage * IST * GC + (k + b) * GC
                pltpu.async_copy(bufs[b], out.at[pl.ds(base, GC)], semw[b])

    wait_write(0)
    wait_write(1)


def _gather_body(row2, col2, tsrc, tdst, gsrc, gdst, idx, buf0, buf1, tab,
                 semw0, semw1):
    cid = lax.axis_index("c")
    sid = lax.axis_index("s")

    @pl.when(cid == 0)
    def _():
        pltpu.sync_copy(tsrc.at[pl.ds(sid * NTS, NTS)],
                        tab.at[pl.ds(sid * NTS, NTS)])

    @pl.when(cid == 1)
    def _():
        pltpu.sync_copy(tdst.at[pl.ds(sid * NTS, NTS)],
                        tab.at[pl.ds(sid * NTS, NTS)])

    plsc.subcore_barrier()

    @pl.when(cid == 0)
    def _():
        _gather_pipe(row2, gsrc, sid, idx, buf0, buf1, tab, semw0, semw1)

    @pl.when(cid == 1)
    def _():
        _gather_pipe(col2, gdst, sid, idx, buf0, buf1, tab, semw0, semw1)


@functools.cache
def _gather_call():
    return pl.kernel(
        _gather_body,
        out_type=(jax.ShapeDtypeStruct((EPAD, TW), F32),
                  jax.ShapeDtypeStruct((EPAD, TW), F32)),
        mesh=_sc_mesh(),
        scratch_types=[
            pltpu.VMEM((IST, GC), jnp.int32),
            pltpu.VMEM((GC, TW), F32),
            pltpu.VMEM((GC, TW), F32),
            pltpu.VMEM_SHARED((NT, TW), F32),
            pltpu.SemaphoreType.DMA,
            pltpu.SemaphoreType.DMA,
        ],
    )


def _gather_pallas(row2, col2, tsrc, tdst):
    return _gather_call()(row2, col2, tsrc, tdst)


def _scatter_body(row2, pay, zeros, part, idx2, buf0, buf1, acc, seml0,
                  seml1):
    cid = lax.axis_index("c")
    sid = lax.axis_index("s")
    wid = sid * 2 + cid
    bufs = (buf0, buf1)
    seml = (seml0, seml1)

    @pl.when(sid == 0)
    def _():
        pltpu.sync_copy(zeros, acc)

    pltpu.sync_copy(row2.at[pl.ds(wid * NCH, NCH)], idx2)
    plsc.subcore_barrier()

    def issue(c, b):
        pltpu.async_copy(pay.at[pl.ds(wid * EPT + c * GC, GC)], bufs[b],
                         seml[b])

    def wait_load(b):
        pltpu.make_async_copy(pay.at[pl.ds(0, GC)], bufs[b], seml[b]).wait()

    issue(0, 0)

    @pl.loop(0, NCH, step=2)
    def _(i):
        for b in range(2):
            c = i + b
            wait_load(b)

            @pl.when(c + 1 < NCH)
            def _():
                issue(c + 1, 1 - b)

            pltpu.sync_copy(bufs[b], acc.at[idx2.at[c]], add=True)

    plsc.subcore_barrier()

    @pl.when(sid == 0)
    def _():
        pltpu.sync_copy(acc, part.at[cid])


@functools.cache
def _scatter_call():
    return pl.kernel(
        _scatter_body,
        out_type=jax.ShapeDtypeStruct((2, NT, TW), F32),
        mesh=_sc_mesh(),
        scratch_types=[
            pltpu.VMEM((NCH, GC), jnp.int32),
            pltpu.VMEM((GC, TW), F32),
            pltpu.VMEM((GC, TW), F32),
            pltpu.VMEM_SHARED((NT, TW), F32),
            pltpu.SemaphoreType.DMA,
            pltpu.SemaphoreType.DMA,
        ],
    )


def _scatter_pallas(row2, payload, zeros):
    return _scatter_call()(row2, payload, zeros)


def _node0_body(his_ref, wemb_ref, bemb_ref, vel_ref, wsrc_ref, wdst_ref,
                wv1_ref, bv1_ref, wv2t_ref, bv2_ref,
                h_ref, hs_ref, hd_ref, vmod_ref, nvel_ref):
    h = jnp.dot(his_ref[...], wemb_ref[...], preferred_element_type=F32)
    h = h + bemb_ref[...]
    h_ref[...] = h
    hs_ref[...] = jnp.dot(h, wsrc_ref[...], preferred_element_type=F32)
    hd_ref[...] = jnp.dot(h, wdst_ref[...], preferred_element_type=F32)
    q = _silu(jnp.dot(h, wv1_ref[...], preferred_element_type=F32)
              + bv1_ref[...])
    vmod_ref[...] = (jnp.sum(q * wv2t_ref[...], axis=1, keepdims=True)
                     + bv2_ref[...])
    v = vel_ref[...]
    nrm = jnp.sqrt(jnp.sum(v * v, axis=1, keepdims=True)) + 1e-8
    nvel_ref[...] = v / nrm


NB = 1000
NNB = N // NB


def _node0_pallas(his, vel, wemb, bemb, wsrc, wdst, wv1, bv1, wv2t, bv2):
    blk = lambda c: pl.BlockSpec((NB, c), lambda i: (i, 0))
    rep = lambda r, c: pl.BlockSpec((r, c), lambda i: (0, 0))
    return pl.pallas_call(
        _node0_body,
        grid=(NNB,),
        in_specs=[
            blk(128), rep(128, H), rep(1, H), blk(3), rep(H, H), rep(H, H),
            rep(H, H), rep(1, H), rep(1, H), rep(1, 1),
        ],
        out_specs=(blk(H), blk(H), blk(H), blk(1), blk(3)),
        out_shape=(
            jax.ShapeDtypeStruct((N, H), F32),
            jax.ShapeDtypeStruct((N, H), F32),
            jax.ShapeDtypeStruct((N, H), F32),
            jax.ShapeDtypeStruct((N, 1), F32),
            jax.ShapeDtypeStruct((N, 3), F32),
        ),
    )(his, wemb, bemb, vel, wsrc, wdst, wv1, bv1, wv2t, bv2)


def _eaproj_body(ea_ref, wea_ref, be1_ref, out_ref):
    out_ref[...] = (jnp.dot(ea_ref[...], wea_ref[...],
                            preferred_element_type=F32) + be1_ref[...])


def _eaproj_pallas(ea_pad, wea, be1):
    return pl.pallas_call(
        _eaproj_body,
        grid=(NEB,),
        in_specs=[
            pl.BlockSpec((EB, D_EDGE), lambda i: (i, 0)),
            pl.BlockSpec((D_EDGE, H), lambda i: (0, 0)),
            pl.BlockSpec((1, H), lambda i: (0, 0)),
        ],
        out_specs=pl.BlockSpec((EB, H), lambda i: (i, 0)),
        out_shape=jax.ShapeDtypeStruct((EPAD, H), F32),
    )(ea_pad, wea, be1)


def _edge_body(gs_ref, gd_ref, eab_ref, radw_ref, we2_ref, be2_ref,
               wc1_ref, bc1_ref, wc2t_ref, out_ref):
    gs = gs_ref[...]
    gd = gd_ref[...]
    cd = gs[:, H:H + 16] - gd[:, H:H + 16]
    radial = jnp.sum(cd * cd, axis=1, keepdims=True)
    pre1 = gs[:, 0:H] + gd[:, 0:H] + eab_ref[...] + radial * radw_ref[...]
    m1 = _silu(pre1)
    m = _silu(jnp.dot(m1, we2_ref[...], preferred_element_type=F32)
              + be2_ref[...])
    q = _silu(jnp.dot(m, wc1_ref[...], preferred_element_type=F32)
              + bc1_ref[...])
    cm = jnp.sum(q * wc2t_ref[...], axis=1, keepdims=True)
    t16 = cd * cm
    lane = lax.broadcasted_iota(jnp.int32, t16.shape, 1)
    t16 = jnp.where(lane == 3, 1.0, t16)
    out_ref[...] = jnp.concatenate(
        [t16, m, jnp.zeros((t16.shape[0], TW - 16 - H), F32)], axis=1)


def _edge_pallas(gsrc, gdst, eab, radw, we2, be2, wc1, bc1, wc2t):
    full = lambda s: pl.BlockSpec(s, lambda i: tuple(0 for _ in s))
    return pl.pallas_call(
        _edge_body,
        grid=(NEB,),
        in_specs=[
            pl.BlockSpec((EB, TW), lambda i: (i, 0)),
            pl.BlockSpec((EB, TW), lambda i: (i, 0)),
            pl.BlockSpec((EB, H), lambda i: (i, 0)),
            full((1, H)),
            full((H, H)),
            full((1, H)),
            full((H, H)),
            full((1, H)),
            full((1, H)),
        ],
        out_specs=pl.BlockSpec((EB, TW), lambda i: (i, 0)),
        out_shape=jax.ShapeDtypeStruct((EPAD, TW), F32),
    )(gsrc, gdst, eab, radw, we2, be2, wc1, bc1, wc2t)


def _node_body(p_ref, h_ref, x_ref, v_ref, nvel_ref, vmod_ref,
               wnh_ref, wnm_ref, bn1_ref, wn2_ref, bn2_ref,
               wsrc_ref, wdst_ref, wv1_ref, bv1_ref, wv2t_ref, bv2_ref,
               hn_ref, xn_ref, vn_ref, hs_ref, hd_ref, vmodn_ref):
    ps = p_ref[0] + p_ref[1]
    cnt = jnp.maximum(ps[:, 3:4], 1.0)
    agg = ps[:, 0:3] / cnt
    v = v_ref[...] + agg + vmod_ref[...] * nvel_ref[...]
    vn_ref[...] = v
    xn_ref[...] = x_ref[...] + v
    magg = ps[:, 16:80]
    h = h_ref[...]
    hmid = _silu(jnp.dot(h, wnh_ref[...], preferred_element_type=F32)
                 + jnp.dot(magg, wnm_ref[...], preferred_element_type=F32)
                 + bn1_ref[...])
    hn = h + jnp.dot(hmid, wn2_ref[...], preferred_element_type=F32) \
        + bn2_ref[...]
    hn_ref[...] = hn
    hs_ref[...] = jnp.dot(hn, wsrc_ref[...], preferred_element_type=F32)
    hd_ref[...] = jnp.dot(hn, wdst_ref[...], preferred_element_type=F32)
    q = _silu(jnp.dot(hn, wv1_ref[...], preferred_element_type=F32)
              + bv1_ref[...])
    vmodn_ref[...] = (jnp.sum(q * wv2t_ref[...], axis=1, keepdims=True)
                      + bv2_ref[...])


def _node_pallas(partials, h, x, v, nvel, vmod, wnh, wnm, bn1, wn2, bn2,
                 wsrc, wdst, wv1, bv1, wv2t, bv2):
    blk = lambda c: pl.BlockSpec((NB, c), lambda i: (i, 0))
    rep = lambda r, c: pl.BlockSpec((r, c), lambda i: (0, 0))
    return pl.pallas_call(
        _node_body,
        grid=(NNB,),
        in_specs=[
            pl.BlockSpec((2, NB, TW), lambda i: (0, i, 0)),
            blk(H), blk(3), blk(3), blk(3), blk(1),
            rep(H, H), rep(H, H), rep(1, H), rep(H, H), rep(1, H),
            rep(H, H), rep(H, H), rep(H, H), rep(1, H), rep(1, H), rep(1, 1),
        ],
        out_specs=(blk(H), blk(3), blk(3), blk(H), blk(H), blk(1)),
        out_shape=(
            jax.ShapeDtypeStruct((N, H), F32),
            jax.ShapeDtypeStruct((N, 3), F32),
            jax.ShapeDtypeStruct((N, 3), F32),
            jax.ShapeDtypeStruct((N, H), F32),
            jax.ShapeDtypeStruct((N, H), F32),
            jax.ShapeDtypeStruct((N, 1), F32),
        ),
    )(partials, h, x, v, nvel, vmod, wnh, wnm, bn1, wn2, bn2,
      wsrc, wdst, wv1, bv1, wv2t, bv2)


def _table(hproj, x):
    t = jnp.concatenate([hproj, x, jnp.zeros((N, TW - H - 3), F32)], axis=1)
    return jnp.pad(t, ((0, NT - N), (0, 0)))


def kernel(his, loc, edges, vel, edge_attr, W_emb, b_emb, We1, be1, We2, be2,
           Wn1, bn1, Wn2, bn2, Wc1, bc1, Wc2, Wv1, bv1, Wv2, bv2):
    row = edges[0]
    col = edges[1]
    pad = jnp.full((EPAD - E,), N, jnp.int32)
    row2 = jnp.concatenate([row, pad]).reshape(EPAD // GC, GC)
    col2 = jnp.concatenate([col, pad]).reshape(EPAD // GC, GC)
    ea_pad = jnp.pad(edge_attr, ((0, EPAD - E), (0, 0)))
    zeros = jnp.zeros((NT, TW), F32)

    wsrc = We1[0:H]
    wdst = We1[H:2 * H]
    radw = We1[2 * H:2 * H + 1]
    wea = We1[2 * H + 1:]
    wnh = Wn1[0:H]
    wnm = Wn1[H:2 * H]
    b_emb2 = b_emb.reshape(1, H)
    be1_2 = be1.reshape(1, H)
    be2_2 = be2.reshape(1, H)
    bn1_2 = bn1.reshape(1, H)
    bn2_2 = bn2.reshape(1, H)
    bc1_2 = bc1.reshape(1, H)
    bv1_2 = bv1.reshape(1, H)
    wc2t = Wc2.reshape(1, H)
    wv2t = Wv2.reshape(1, H)
    bv2_2 = bv2.reshape(1, 1)

    h, hs, hd, vmod, nvel = _node0_pallas(
        his, vel, W_emb, b_emb2, wsrc, wdst, Wv1, bv1_2, wv2t, bv2_2)
    eab = _eaproj_pallas(ea_pad, wea, be1_2)

    x = loc
    v = vel
    for _ in range(4):
        tsrc = _table(hs, x)
        tdst = _table(hd, x)
        gsrc, gdst = _gather_pallas(row2, col2, tsrc, tdst)
        payload = _edge_pallas(gsrc, gdst, eab, radw, We2, be2_2,
                               Wc1, bc1_2, wc2t)
        partials = _scatter_pallas(row2, payload, zeros)
        h, x, v, hs, hd, vmod = _node_pallas(
            partials, h, x, v, nvel, vmod, wnh, wnm, bn1_2, Wn2, bn2_2,
            wsrc, wdst, Wv1, bv1_2, wv2t, bv2_2)
    return x, h, v


# speedup vs baseline: 2.2625x; 1.1399x over previous
import functools

import jax
import jax.numpy as jnp
from jax import lax
from jax.experimental import pallas as pl
from jax.experimental.pallas import tpu as pltpu
from jax.experimental.pallas import tpu_sc as plsc

N = 10000
E = 320000
H = 64
D_EDGE = 16

NT = 10240
TW = 128
NW = 32
EPAD = 327680
EPT = EPAD // NW
GC = 128
NCH = EPT // GC
SB = 256
NSB = EPT // SB
EB = 2048
NEB = EPAD // EB

F32 = jnp.float32
BF16 = jnp.bfloat16

@functools.cache
def _sc_mesh():
    return plsc.VectorSubcoreMesh(core_axis_name="c", subcore_axis_name="s",
                                  num_cores=2, num_subcores=16)


def _silu(x):
    return x * jax.nn.sigmoid(x)


EH = EPAD // 2
EHG = EH // 16
IST = EHG // GC
NTS = NT // 16


def _gather_pipe(half, idx2, out, sid, idx, buf0, buf1, tab, semw0, semw1):
    bufs = (buf0, buf1)
    semw = (semw0, semw1)

    def wait_write(b):
        pltpu.make_async_copy(bufs[b], out.at[pl.ds(0, GC)], semw[b]).wait()

    pltpu.sync_copy(
        idx2.at[pl.ds(half * (EH // GC) + sid * IST, IST)], idx)

    @pl.loop(0, IST, step=2)
    def _(k):
        for b in range(2):
            @pl.when(k + b >= 2)
            def _():
                wait_write(b)

            pltpu.sync_copy(tab.at[idx.at[k + b]], bufs[b])
            base = sid * EHG + (k + b) * GC
            pltpu.async_copy(bufs[b], out.at[pl.ds(base, GC)], semw[b])

    wait_write(0)
    wait_write(1)


def _make_gather_body(half):
    def body(row2, col2, tsrc, tdst, gsrc, gdst, idx, buf0, buf1, tab,
             semw0, semw1):
        cid = lax.axis_index("c")
        sid = lax.axis_index("s")

        @pl.when(cid == 0)
        def _():
            pltpu.sync_copy(tsrc.at[pl.ds(sid * NTS, NTS)],
                            tab.at[pl.ds(sid * NTS, NTS)])

        @pl.when(cid == 1)
        def _():
            pltpu.sync_copy(tdst.at[pl.ds(sid * NTS, NTS)],
                            tab.at[pl.ds(sid * NTS, NTS)])

        plsc.subcore_barrier()

        @pl.when(cid == 0)
        def _():
            _gather_pipe(half, row2, gsrc, sid, idx, buf0, buf1, tab,
                         semw0, semw1)

        @pl.when(cid == 1)
        def _():
            _gather_pipe(half, col2, gdst, sid, idx, buf0, buf1, tab,
                         semw0, semw1)

    return body


@functools.cache
def _gather_call(half):
    return pl.kernel(
        _make_gather_body(half),
        out_type=(jax.ShapeDtypeStruct((EH, TW), F32),
                  jax.ShapeDtypeStruct((EH, TW), F32)),
        mesh=_sc_mesh(),
        scratch_types=[
            pltpu.VMEM((IST, GC), jnp.int32),
            pltpu.VMEM((GC, TW), F32),
            pltpu.VMEM((GC, TW), F32),
            pltpu.VMEM_SHARED((NT, TW), F32),
            pltpu.SemaphoreType.DMA,
            pltpu.SemaphoreType.DMA,
        ],
    )


def _gather_pallas(half, row2, col2, tsrc, tdst):
    return _gather_call(half)(row2, col2, tsrc, tdst)


EHS = EH // 32
NCHH = EHS // GC


def _make_scatter_body(half):
    def body(row2, pay, zeros, part, idx2, buf0, buf1, acc, seml0, seml1):
        cid = lax.axis_index("c")
        sid = lax.axis_index("s")
        wid = sid * 2 + cid
        bufs = (buf0, buf1)
        seml = (seml0, seml1)

        @pl.when(sid == 0)
        def _():
            pltpu.sync_copy(zeros, acc)

        pltpu.sync_copy(
            row2.at[pl.ds(half * (EH // GC) + wid * NCHH, NCHH)], idx2)
        plsc.subcore_barrier()

        def issue(c, b):
            pltpu.async_copy(pay.at[pl.ds(wid * EHS + c * GC, GC)], bufs[b],
                             seml[b])

        def wait_load(b):
            pltpu.make_async_copy(pay.at[pl.ds(0, GC)], bufs[b],
                                  seml[b]).wait()

        issue(0, 0)

        @pl.loop(0, NCHH, step=2)
        def _(i):
            for b in range(2):
                c = i + b
                wait_load(b)

                @pl.when(c + 1 < NCHH)
                def _():
                    issue(c + 1, 1 - b)

                pltpu.sync_copy(bufs[b], acc.at[idx2.at[c]], add=True)

        plsc.subcore_barrier()

        @pl.when(sid == 0)
        def _():
            pltpu.sync_copy(acc, part.at[cid])

    return body


@functools.cache
def _scatter_call(half):
    return pl.kernel(
        _make_scatter_body(half),
        out_type=jax.ShapeDtypeStruct((2, NT, TW), F32),
        mesh=_sc_mesh(),
        scratch_types=[
            pltpu.VMEM((NCHH, GC), jnp.int32),
            pltpu.VMEM((GC, TW), F32),
            pltpu.VMEM((GC, TW), F32),
            pltpu.VMEM_SHARED((NT, TW), F32),
            pltpu.SemaphoreType.DMA,
            pltpu.SemaphoreType.DMA,
        ],
    )


def _scatter_pallas(half, row2, payload, zeros):
    return _scatter_call(half)(row2, payload, zeros)


def _node0_body(his_ref, wemb_ref, bemb_ref, vel_ref, wsrc_ref, wdst_ref,
                wv1_ref, bv1_ref, wv2t_ref, bv2_ref,
                h_ref, hs_ref, hd_ref, vmod_ref, nvel_ref):
    h = jnp.dot(his_ref[...], wemb_ref[...], preferred_element_type=F32)
    h = h + bemb_ref[...]
    h_ref[...] = h
    hs_ref[...] = jnp.dot(h, wsrc_ref[...], preferred_element_type=F32)
    hd_ref[...] = jnp.dot(h, wdst_ref[...], preferred_element_type=F32)
    q = _silu(jnp.dot(h, wv1_ref[...], preferred_element_type=F32)
              + bv1_ref[...])
    vmod_ref[...] = (jnp.sum(q * wv2t_ref[...], axis=1, keepdims=True)
                     + bv2_ref[...])
    v = vel_ref[...]
    nrm = jnp.sqrt(jnp.sum(v * v, axis=1, keepdims=True)) + 1e-8
    nvel_ref[...] = v / nrm


NB = 1000
NNB = N // NB


def _node0_pallas(his, vel, wemb, bemb, wsrc, wdst, wv1, bv1, wv2t, bv2):
    blk = lambda c: pl.BlockSpec((NB, c), lambda i: (i, 0))
    rep = lambda r, c: pl.BlockSpec((r, c), lambda i: (0, 0))
    return pl.pallas_call(
        _node0_body,
        grid=(NNB,),
        in_specs=[
            blk(128), rep(128, H), rep(1, H), blk(3), rep(H, H), rep(H, H),
            rep(H, H), rep(1, H), rep(1, H), rep(1, 1),
        ],
        out_specs=(blk(H), blk(H), blk(H), blk(1), blk(3)),
        out_shape=(
            jax.ShapeDtypeStruct((N, H), F32),
            jax.ShapeDtypeStruct((N, H), F32),
            jax.ShapeDtypeStruct((N, H), F32),
            jax.ShapeDtypeStruct((N, 1), F32),
            jax.ShapeDtypeStruct((N, 3), F32),
        ),
    )(his, wemb, bemb, vel, wsrc, wdst, wv1, bv1, wv2t, bv2)


def _eaproj_body(ea_ref, wea_ref, be1_ref, out_ref):
    out_ref[...] = (jnp.dot(ea_ref[...], wea_ref[...],
                            preferred_element_type=F32)
                    + be1_ref[...]).astype(BF16)


def _eaproj_pallas(ea_pad, wea, be1):
    return pl.pallas_call(
        _eaproj_body,
        grid=(NEB,),
        in_specs=[
            pl.BlockSpec((EB, D_EDGE), lambda i: (i, 0)),
            pl.BlockSpec((D_EDGE, H), lambda i: (0, 0)),
            pl.BlockSpec((1, H), lambda i: (0, 0)),
        ],
        out_specs=pl.BlockSpec((EB, H), lambda i: (i, 0)),
        out_shape=jax.ShapeDtypeStruct((EPAD, H), BF16),
    )(ea_pad, wea, be1)


def _edge_body(gs_ref, gd_ref, eab_ref, radw_ref, we2_ref, be2_ref,
               wc1_ref, bc1_ref, wc2t_ref, out_ref):
    gs = gs_ref[...].astype(F32)
    gd = gd_ref[...].astype(F32)
    cd = gs[:, H:H + 16] - gd[:, H:H + 16]
    radial = jnp.sum(cd * cd, axis=1, keepdims=True)
    pre1 = (gs[:, 0:H] + gd[:, 0:H] + eab_ref[...].astype(F32)
            + radial * radw_ref[...])
    m1 = _silu(pre1)
    m = _silu(jnp.dot(m1, we2_ref[...], preferred_element_type=F32)
              + be2_ref[...])
    q = _silu(jnp.dot(m, wc1_ref[...], preferred_element_type=F32)
              + bc1_ref[...])
    cm = jnp.sum(q * wc2t_ref[...], axis=1, keepdims=True)
    t16 = cd * cm
    lane = lax.broadcasted_iota(jnp.int32, t16.shape, 1)
    t16 = jnp.where(lane == 3, 1.0, t16)
    out_ref[...] = jnp.concatenate(
        [t16, m, jnp.zeros((t16.shape[0], TW - 16 - H), F32)], axis=1)


def _edge_pallas(half, gsrc, gdst, eab, radw, we2, be2, wc1, bc1, wc2t):
    full = lambda s: pl.BlockSpec(s, lambda i: tuple(0 for _ in s))
    off = half * (EH // EB)
    return pl.pallas_call(
        _edge_body,
        grid=(EH // EB,),
        in_specs=[
            pl.BlockSpec((EB, TW), lambda i: (i, 0)),
            pl.BlockSpec((EB, TW), lambda i: (i, 0)),
            pl.BlockSpec((EB, H), lambda i: (i + off, 0)),
            full((1, H)),
            full((H, H)),
            full((1, H)),
            full((H, H)),
            full((1, H)),
            full((1, H)),
        ],
        out_specs=pl.BlockSpec((EB, TW), lambda i: (i, 0)),
        out_shape=jax.ShapeDtypeStruct((EH, TW), F32),
    )(gsrc, gdst, eab, radw, we2, be2, wc1, bc1, wc2t)


def _node_body(p_ref, q_ref, h_ref, x_ref, v_ref, nvel_ref, vmod_ref,
               wnh_ref, wnm_ref, bn1_ref, wn2_ref, bn2_ref,
               wsrc_ref, wdst_ref, wv1_ref, bv1_ref, wv2t_ref, bv2_ref,
               hn_ref, xn_ref, vn_ref, hs_ref, hd_ref, vmodn_ref):
    ps = (p_ref[0] + p_ref[1]) + (q_ref[0] + q_ref[1])
    cnt = jnp.maximum(ps[:, 3:4], 1.0)
    agg = ps[:, 0:3] / cnt
    v = v_ref[...] + agg + vmod_ref[...] * nvel_ref[...]
    vn_ref[...] = v
    xn_ref[...] = x_ref[...] + v
    magg = ps[:, 16:80]
    h = h_ref[...]
    hmid = _silu(jnp.dot(h, wnh_ref[...], preferred_element_type=F32)
                 + jnp.dot(magg, wnm_ref[...], preferred_element_type=F32)
                 + bn1_ref[...])
    hn = h + jnp.dot(hmid, wn2_ref[...], preferred_element_type=F32) \
        + bn2_ref[...]
    hn_ref[...] = hn
    hs_ref[...] = jnp.dot(hn, wsrc_ref[...], preferred_element_type=F32)
    hd_ref[...] = jnp.dot(hn, wdst_ref[...], preferred_element_type=F32)
    q = _silu(jnp.dot(hn, wv1_ref[...], preferred_element_type=F32)
              + bv1_ref[...])
    vmodn_ref[...] = (jnp.sum(q * wv2t_ref[...], axis=1, keepdims=True)
                      + bv2_ref[...])


def _node_pallas(pa, pb, h, x, v, nvel, vmod, wnh, wnm, bn1, wn2, bn2,
                 wsrc, wdst, wv1, bv1, wv2t, bv2):
    blk = lambda c: pl.BlockSpec((NB, c), lambda i: (i, 0))
    rep = lambda r, c: pl.BlockSpec((r, c), lambda i: (0, 0))
    pspec = pl.BlockSpec((2, NB, TW), lambda i: (0, i, 0))
    return pl.pallas_call(
        _node_body,
        grid=(NNB,),
        in_specs=[
            pspec, pspec,
            blk(H), blk(3), blk(3), blk(3), blk(1),
            rep(H, H), rep(H, H), rep(1, H), rep(H, H), rep(1, H),
            rep(H, H), rep(H, H), rep(H, H), rep(1, H), rep(1, H), rep(1, 1),
        ],
        out_specs=(blk(H), blk(3), blk(3), blk(H), blk(H), blk(1)),
        out_shape=(
            jax.ShapeDtypeStruct((N, H), F32),
            jax.ShapeDtypeStruct((N, 3), F32),
            jax.ShapeDtypeStruct((N, 3), F32),
            jax.ShapeDtypeStruct((N, H), F32),
            jax.ShapeDtypeStruct((N, H), F32),
            jax.ShapeDtypeStruct((N, 1), F32),
        ),
    )(pa, pb, h, x, v, nvel, vmod, wnh, wnm, bn1, wn2, bn2,
      wsrc, wdst, wv1, bv1, wv2t, bv2)


def _table(hproj, x):
    t = jnp.concatenate([hproj, x, jnp.zeros((N, TW - H - 3), F32)], axis=1)
    return jnp.pad(t, ((0, NT - N), (0, 0)))


def kernel(his, loc, edges, vel, edge_attr, W_emb, b_emb, We1, be1, We2, be2,
           Wn1, bn1, Wn2, bn2, Wc1, bc1, Wc2, Wv1, bv1, Wv2, bv2):
    row = edges[0]
    col = edges[1]
    pad = jnp.full((EPAD - E,), N, jnp.int32)
    row2 = jnp.concatenate([row, pad]).reshape(EPAD // GC, GC)
    col2 = jnp.concatenate([col, pad]).reshape(EPAD // GC, GC)
    ea_pad = jnp.pad(edge_attr, ((0, EPAD - E), (0, 0)))
    zeros = jnp.zeros((NT, TW), F32)

    wsrc = We1[0:H]
    wdst = We1[H:2 * H]
    radw = We1[2 * H:2 * H + 1]
    wea = We1[2 * H + 1:]
    wnh = Wn1[0:H]
    wnm = Wn1[H:2 * H]
    b_emb2 = b_emb.reshape(1, H)
    be1_2 = be1.reshape(1, H)
    be2_2 = be2.reshape(1, H)
    bn1_2 = bn1.reshape(1, H)
    bn2_2 = bn2.reshape(1, H)
    bc1_2 = bc1.reshape(1, H)
    bv1_2 = bv1.reshape(1, H)
    wc2t = Wc2.reshape(1, H)
    wv2t = Wv2.reshape(1, H)
    bv2_2 = bv2.reshape(1, 1)

    h, hs, hd, vmod, nvel = _node0_pallas(
        his, vel, W_emb, b_emb2, wsrc, wdst, Wv1, bv1_2, wv2t, bv2_2)
    eab = _eaproj_pallas(ea_pad, wea, be1_2)

    x = loc
    v = vel
    for _ in range(4):
        tsrc = _table(hs, x)
        tdst = _table(hd, x)
        ga = _gather_pallas(0, row2, col2, tsrc, tdst)
        gb = _gather_pallas(1, row2, col2, tsrc, tdst)
        paya = _edge_pallas(0, ga[0], ga[1], eab, radw, We2, be2_2,
                            Wc1, bc1_2, wc2t)
        payb = _edge_pallas(1, gb[0], gb[1], eab, radw, We2, be2_2,
                            Wc1, bc1_2, wc2t)
        pa = _scatter_pallas(0, row2, paya, zeros)
        pb = _scatter_pallas(1, row2, payb, zeros)
        h, x, v, hs, hd, vmod = _node_pallas(
            pa, pb, h, x, v, nvel, vmod, wnh, wnm, bn1_2, Wn2, bn2_2,
            wsrc, wdst, Wv1, bv1_2, wv2t, bv2_2)
    return x, h, v


# speedup vs baseline: 2.2735x; 1.0049x over previous
import functools

import jax
import jax.numpy as jnp
from jax import lax
from jax.experimental import pallas as pl
from jax.experimental.pallas import tpu as pltpu
from jax.experimental.pallas import tpu_sc as plsc

N = 10000
E = 320000
H = 64
D_EDGE = 16

NT = 10240
TW = 128
NW = 32
EPAD = 327680
EPT = EPAD // NW
GC = 128
NCH = EPT // GC
SB = 256
NSB = EPT // SB
EB = 2048
NEB = EPAD // EB

F32 = jnp.float32
BF16 = jnp.bfloat16

@functools.cache
def _sc_mesh():
    return plsc.VectorSubcoreMesh(core_axis_name="c", subcore_axis_name="s",
                                  num_cores=2, num_subcores=16)


def _silu(x):
    return x * jax.nn.sigmoid(x)


EH = EPAD // 2
EHG = EH // 16
IST = EHG // GC
NTS = NT // 16


def _gather_pipe(half, idx2, out, sid, idx, buf0, buf1, tab, semg0, semg1,
                 semw0, semw1):
    bufs = (buf0, buf1)
    semg = (semg0, semg1)
    semw = (semw0, semw1)

    def issue_gather(c, b):
        pltpu.async_copy(tab.at[idx.at[c]], bufs[b], semg[b])

    def wait_gather(b):
        pltpu.make_async_copy(tab.at[idx.at[0]], bufs[b], semg[b]).wait()

    def wait_write(b):
        pltpu.make_async_copy(bufs[b], out.at[pl.ds(0, GC)], semw[b]).wait()

    pltpu.sync_copy(
        idx2.at[pl.ds(half * (EH // GC) + sid * IST, IST)], idx)
    issue_gather(0, 0)

    @pl.loop(0, IST, step=2)
    def _(k):
        for b in range(2):
            c = k + b

            @pl.when(c + 1 < IST)
            def _():
                @pl.when(c >= 1)
                def _():
                    wait_write(1 - b)

                issue_gather(c + 1, 1 - b)

            wait_gather(b)
            pltpu.async_copy(bufs[b], out.at[pl.ds(sid * EHG + c * GC, GC)],
                             semw[b])

    wait_write(0)
    wait_write(1)


def _make_gather_body(half):
    def body(row2, col2, tsrc, tdst, gsrc, gdst, idx, buf0, buf1, tab,
             semg0, semg1, semw0, semw1):
        cid = lax.axis_index("c")
        sid = lax.axis_index("s")

        @pl.when(cid == 0)
        def _():
            pltpu.sync_copy(tsrc.at[pl.ds(sid * NTS, NTS)],
                            tab.at[pl.ds(sid * NTS, NTS)])

        @pl.when(cid == 1)
        def _():
            pltpu.sync_copy(tdst.at[pl.ds(sid * NTS, NTS)],
                            tab.at[pl.ds(sid * NTS, NTS)])

        plsc.subcore_barrier()

        @pl.when(cid == 0)
        def _():
            _gather_pipe(half, row2, gsrc, sid, idx, buf0, buf1, tab,
                         semg0, semg1, semw0, semw1)

        @pl.when(cid == 1)
        def _():
            _gather_pipe(half, col2, gdst, sid, idx, buf0, buf1, tab,
                         semg0, semg1, semw0, semw1)

    return body


@functools.cache
def _gather_call(half):
    return pl.kernel(
        _make_gather_body(half),
        out_type=(jax.ShapeDtypeStruct((EH, TW), F32),
                  jax.ShapeDtypeStruct((EH, TW), F32)),
        mesh=_sc_mesh(),
        scratch_types=[
            pltpu.VMEM((IST, GC), jnp.int32),
            pltpu.VMEM((GC, TW), F32),
            pltpu.VMEM((GC, TW), F32),
            pltpu.VMEM_SHARED((NT, TW), F32),
            pltpu.SemaphoreType.DMA,
            pltpu.SemaphoreType.DMA,
            pltpu.SemaphoreType.DMA,
            pltpu.SemaphoreType.DMA,
        ],
    )


def _gather_pallas(half, row2, col2, tsrc, tdst):
    return _gather_call(half)(row2, col2, tsrc, tdst)


EHS = EH // 32
NCHH = EHS // GC


def _make_scatter_body(half):
    def body(row2, pay, zeros, part, idx2, buf0, buf1, acc, seml0, seml1,
             sems0, sems1):
        cid = lax.axis_index("c")
        sid = lax.axis_index("s")
        wid = sid * 2 + cid
        bufs = (buf0, buf1)
        seml = (seml0, seml1)
        sems = (sems0, sems1)

        @pl.when(sid == 0)
        def _():
            pltpu.sync_copy(zeros, acc)

        pltpu.sync_copy(
            row2.at[pl.ds(half * (EH // GC) + wid * NCHH, NCHH)], idx2)
        plsc.subcore_barrier()

        def issue_load(c, b):
            pltpu.async_copy(pay.at[pl.ds(wid * EHS + c * GC, GC)], bufs[b],
                             seml[b])

        def wait_load(b):
            pltpu.make_async_copy(pay.at[pl.ds(0, GC)], bufs[b],
                                  seml[b]).wait()

        def wait_scat(b):
            pltpu.make_async_copy(bufs[b], acc.at[idx2.at[0]],
                                  sems[b]).wait()

        issue_load(0, 0)

        @pl.loop(0, NCHH, step=2)
        def _(i):
            for b in range(2):
                c = i + b
                wait_load(b)
                pltpu.async_copy(bufs[b], acc.at[idx2.at[c]], sems[b],
                                 add=True)

                @pl.when(c + 1 < NCHH)
                def _():
                    @pl.when(c >= 1)
                    def _():
                        wait_scat(1 - b)

                    issue_load(c + 1, 1 - b)

        wait_scat(0)
        wait_scat(1)
        plsc.subcore_barrier()

        @pl.when(sid == 0)
        def _():
            pltpu.sync_copy(acc, part.at[cid])

    return body


@functools.cache
def _scatter_call(half):
    return pl.kernel(
        _make_scatter_body(half),
        out_type=jax.ShapeDtypeStruct((2, NT, TW), F32),
        mesh=_sc_mesh(),
        scratch_types=[
            pltpu.VMEM((NCHH, GC), jnp.int32),
            pltpu.VMEM((GC, TW), F32),
            pltpu.VMEM((GC, TW), F32),
            pltpu.VMEM_SHARED((NT, TW), F32),
            pltpu.SemaphoreType.DMA,
            pltpu.SemaphoreType.DMA,
            pltpu.SemaphoreType.DMA,
            pltpu.SemaphoreType.DMA,
        ],
    )


def _scatter_pallas(half, row2, payload, zeros):
    return _scatter_call(half)(row2, payload, zeros)


def _node0_body(his_ref, wemb_ref, bemb_ref, vel_ref, wsrc_ref, wdst_ref,
                wv1_ref, bv1_ref, wv2t_ref, bv2_ref,
                h_ref, hs_ref, hd_ref, vmod_ref, nvel_ref):
    h = jnp.dot(his_ref[...], wemb_ref[...], preferred_element_type=F32)
    h = h + bemb_ref[...]
    h_ref[...] = h
    hs_ref[...] = jnp.dot(h, wsrc_ref[...], preferred_element_type=F32)
    hd_ref[...] = jnp.dot(h, wdst_ref[...], preferred_element_type=F32)
    q = _silu(jnp.dot(h, wv1_ref[...], preferred_element_type=F32)
              + bv1_ref[...])
    vmod_ref[...] = (jnp.sum(q * wv2t_ref[...], axis=1, keepdims=True)
                     + bv2_ref[...])
    v = vel_ref[...]
    nrm = jnp.sqrt(jnp.sum(v * v, axis=1, keepdims=True)) + 1e-8
    nvel_ref[...] = v / nrm


NB = 1000
NNB = N // NB


def _node0_pallas(his, vel, wemb, bemb, wsrc, wdst, wv1, bv1, wv2t, bv2):
    blk = lambda c: pl.BlockSpec((NB, c), lambda i: (i, 0))
    rep = lambda r, c: pl.BlockSpec((r, c), lambda i: (0, 0))
    return pl.pallas_call(
        _node0_body,
        grid=(NNB,),
        in_specs=[
            blk(128), rep(128, H), rep(1, H), blk(3), rep(H, H), rep(H, H),
            rep(H, H), rep(1, H), rep(1, H), rep(1, 1),
        ],
        out_specs=(blk(H), blk(H), blk(H), blk(1), blk(3)),
        out_shape=(
            jax.ShapeDtypeStruct((N, H), F32),
            jax.ShapeDtypeStruct((N, H), F32),
            jax.ShapeDtypeStruct((N, H), F32),
            jax.ShapeDtypeStruct((N, 1), F32),
            jax.ShapeDtypeStruct((N, 3), F32),
        ),
    )(his, wemb, bemb, vel, wsrc, wdst, wv1, bv1, wv2t, bv2)


def _eaproj_body(ea_ref, wea_ref, be1_ref, out_ref):
    out_ref[...] = (jnp.dot(ea_ref[...], wea_ref[...],
                            preferred_element_type=F32)
                    + be1_ref[...]).astype(BF16)


def _eaproj_pallas(ea_pad, wea, be1):
    return pl.pallas_call(
        _eaproj_body,
        grid=(NEB,),
        in_specs=[
            pl.BlockSpec((EB, D_EDGE), lambda i: (i, 0)),
            pl.BlockSpec((D_EDGE, H), lambda i: (0, 0)),
            pl.BlockSpec((1, H), lambda i: (0, 0)),
        ],
        out_specs=pl.BlockSpec((EB, H), lambda i: (i, 0)),
        out_shape=jax.ShapeDtypeStruct((EPAD, H), BF16),
    )(ea_pad, wea, be1)


def _edge_body(gs_ref, gd_ref, eab_ref, radw_ref, we2_ref, be2_ref,
               wc1_ref, bc1_ref, wc2t_ref, out_ref):
    gs = gs_ref[...].astype(F32)
    gd = gd_ref[...].astype(F32)
    cd = gs[:, H:H + 16] - gd[:, H:H + 16]
    radial = jnp.sum(cd * cd, axis=1, keepdims=True)
    pre1 = (gs[:, 0:H] + gd[:, 0:H] + eab_ref[...].astype(F32)
            + radial * radw_ref[...])
    m1 = _silu(pre1)
    m = _silu(jnp.dot(m1, we2_ref[...], preferred_element_type=F32)
              + be2_ref[...])
    q = _silu(jnp.dot(m, wc1_ref[...], preferred_element_type=F32)
              + bc1_ref[...])
    cm = jnp.sum(q * wc2t_ref[...], axis=1, keepdims=True)
    t16 = cd * cm
    lane = lax.broadcasted_iota(jnp.int32, t16.shape, 1)
    t16 = jnp.where(lane == 3, 1.0, t16)
    out_ref[...] = jnp.concatenate(
        [t16, m, jnp.zeros((t16.shape[0], TW - 16 - H), F32)], axis=1)


def _edge_pallas(half, gsrc, gdst, eab, radw, we2, be2, wc1, bc1, wc2t):
    full = lambda s: pl.BlockSpec(s, lambda i: tuple(0 for _ in s))
    off = half * (EH // EB)
    return pl.pallas_call(
        _edge_body,
        grid=(EH // EB,),
        in_specs=[
            pl.BlockSpec((EB, TW), lambda i: (i, 0)),
            pl.BlockSpec((EB, TW), lambda i: (i, 0)),
            pl.BlockSpec((EB, H), lambda i: (i + off, 0)),
            full((1, H)),
            full((H, H)),
            full((1, H)),
            full((H, H)),
            full((1, H)),
            full((1, H)),
        ],
        out_specs=pl.BlockSpec((EB, TW), lambda i: (i, 0)),
        out_shape=jax.ShapeDtypeStruct((EH, TW), F32),
    )(gsrc, gdst, eab, radw, we2, be2, wc1, bc1, wc2t)


def _node_body(p_ref, q_ref, h_ref, x_ref, v_ref, nvel_ref, vmod_ref,
               wnh_ref, wnm_ref, bn1_ref, wn2_ref, bn2_ref,
               wsrc_ref, wdst_ref, wv1_ref, bv1_ref, wv2t_ref, bv2_ref,
               hn_ref, xn_ref, vn_ref, hs_ref, hd_ref, vmodn_ref):
    ps = (p_ref[0] + p_ref[1]) + (q_ref[0] + q_ref[1])
    cnt = jnp.maximum(ps[:, 3:4], 1.0)
    agg = ps[:, 0:3] / cnt
    v = v_ref[...] + agg + vmod_ref[...] * nvel_ref[...]
    vn_ref[...] = v
    xn_ref[...] = x_ref[...] + v
    magg = ps[:, 16:80]
    h = h_ref[...]
    hmid = _silu(jnp.dot(h, wnh_ref[...], preferred_element_type=F32)
                 + jnp.dot(magg, wnm_ref[...], preferred_element_type=F32)
                 + bn1_ref[...])
    hn = h + jnp.dot(hmid, wn2_ref[...], preferred_element_type=F32) \
        + bn2_ref[...]
    hn_ref[...] = hn
    hs_ref[...] = jnp.dot(hn, wsrc_ref[...], preferred_element_type=F32)
    hd_ref[...] = jnp.dot(hn, wdst_ref[...], preferred_element_type=F32)
    q = _silu(jnp.dot(hn, wv1_ref[...], preferred_element_type=F32)
              + bv1_ref[...])
    vmodn_ref[...] = (jnp.sum(q * wv2t_ref[...], axis=1, keepdims=True)
                      + bv2_ref[...])


def _node_pallas(pa, pb, h, x, v, nvel, vmod, wnh, wnm, bn1, wn2, bn2,
                 wsrc, wdst, wv1, bv1, wv2t, bv2):
    blk = lambda c: pl.BlockSpec((NB, c), lambda i: (i, 0))
    rep = lambda r, c: pl.BlockSpec((r, c), lambda i: (0, 0))
    pspec = pl.BlockSpec((2, NB, TW), lambda i: (0, i, 0))
    return pl.pallas_call(
        _node_body,
        grid=(NNB,),
        in_specs=[
            pspec, pspec,
            blk(H), blk(3), blk(3), blk(3), blk(1),
            rep(H, H), rep(H, H), rep(1, H), rep(H, H), rep(1, H),
            rep(H, H), rep(H, H), rep(H, H), rep(1, H), rep(1, H), rep(1, 1),
        ],
        out_specs=(blk(H), blk(3), blk(3), blk(H), blk(H), blk(1)),
        out_shape=(
            jax.ShapeDtypeStruct((N, H), F32),
            jax.ShapeDtypeStruct((N, 3), F32),
            jax.ShapeDtypeStruct((N, 3), F32),
            jax.ShapeDtypeStruct((N, H), F32),
            jax.ShapeDtypeStruct((N, H), F32),
            jax.ShapeDtypeStruct((N, 1), F32),
        ),
    )(pa, pb, h, x, v, nvel, vmod, wnh, wnm, bn1, wn2, bn2,
      wsrc, wdst, wv1, bv1, wv2t, bv2)


def _table(hproj, x):
    t = jnp.concatenate([hproj, x, jnp.zeros((N, TW - H - 3), F32)], axis=1)
    return jnp.pad(t, ((0, NT - N), (0, 0)))


def kernel(his, loc, edges, vel, edge_attr, W_emb, b_emb, We1, be1, We2, be2,
           Wn1, bn1, Wn2, bn2, Wc1, bc1, Wc2, Wv1, bv1, Wv2, bv2):
    row = edges[0]
    col = edges[1]
    pad = jnp.full((EPAD - E,), N, jnp.int32)
    row2 = jnp.concatenate([row, pad]).reshape(EPAD // GC, GC)
    col2 = jnp.concatenate([col, pad]).reshape(EPAD // GC, GC)
    ea_pad = jnp.pad(edge_attr, ((0, EPAD - E), (0, 0)))
    zeros = jnp.zeros((NT, TW), F32)

    wsrc = We1[0:H]
    wdst = We1[H:2 * H]
    radw = We1[2 * H:2 * H + 1]
    wea = We1[2 * H + 1:]
    wnh = Wn1[0:H]
    wnm = Wn1[H:2 * H]
    b_emb2 = b_emb.reshape(1, H)
    be1_2 = be1.reshape(1, H)
    be2_2 = be2.reshape(1, H)
    bn1_2 = bn1.reshape(1, H)
    bn2_2 = bn2.reshape(1, H)
    bc1_2 = bc1.reshape(1, H)
    bv1_2 = bv1.reshape(1, H)
    wc2t = Wc2.reshape(1, H)
    wv2t = Wv2.reshape(1, H)
    bv2_2 = bv2.reshape(1, 1)

    h, hs, hd, vmod, nvel = _node0_pallas(
        his, vel, W_emb, b_emb2, wsrc, wdst, Wv1, bv1_2, wv2t, bv2_2)
    eab = _eaproj_pallas(ea_pad, wea, be1_2)

    x = loc
    v = vel
    for _ in range(4):
        tsrc = _table(hs, x)
        tdst = _table(hd, x)
        ga = _gather_pallas(0, row2, col2, tsrc, tdst)
        gb = _gather_pallas(1, row2, col2, tsrc, tdst)
        paya = _edge_pallas(0, ga[0], ga[1], eab, radw, We2, be2_2,
                            Wc1, bc1_2, wc2t)
        payb = _edge_pallas(1, gb[0], gb[1], eab, radw, We2, be2_2,
                            Wc1, bc1_2, wc2t)
        pa = _scatter_pallas(0, row2, paya, zeros)
        pb = _scatter_pallas(1, row2, payb, zeros)
        h, x, v, hs, hd, vmod = _node_pallas(
            pa, pb, h, x, v, nvel, vmod, wnh, wnm, bn1_2, Wn2, bn2_2,
            wsrc, wdst, Wv1, bv1_2, wv2t, bv2_2)
    return x, h, v


# speedup vs baseline: 2.2762x; 1.0012x over previous
import functools

import jax
import jax.numpy as jnp
from jax import lax
from jax.experimental import pallas as pl
from jax.experimental.pallas import tpu as pltpu
from jax.experimental.pallas import tpu_sc as plsc

N = 10000
E = 320000
H = 64
D_EDGE = 16

NT = 10240
TW = 128
NW = 32
EPAD = 327680
EPT = EPAD // NW
GC = 128
NCH = EPT // GC
SB = 256
NSB = EPT // SB
EB = 2048
NEB = EPAD // EB

F32 = jnp.float32
BF16 = jnp.bfloat16

@functools.cache
def _sc_mesh():
    return plsc.VectorSubcoreMesh(core_axis_name="c", subcore_axis_name="s",
                                  num_cores=2, num_subcores=16)


def _silu(x):
    return x * jax.nn.sigmoid(x)


EH = EPAD // 2
EHG = EH // 16
IST = EHG // GC
NTS = NT // 16


def _gather_pipe(half, idx2, out, sid, idx, buf0, buf1, tab, semg0, semg1,
                 semw0, semw1):
    bufs = (buf0, buf1)
    semg = (semg0, semg1)
    semw = (semw0, semw1)

    def issue_gather(c, b):
        pltpu.async_copy(tab.at[idx.at[c]], bufs[b], semg[b])

    def wait_gather(b):
        pltpu.make_async_copy(tab.at[idx.at[0]], bufs[b], semg[b]).wait()

    def wait_write(b):
        pltpu.make_async_copy(bufs[b], out.at[pl.ds(0, GC)], semw[b]).wait()

    pltpu.sync_copy(
        idx2.at[pl.ds(half * (EH // GC) + sid * IST, IST)], idx)
    issue_gather(0, 0)

    @pl.loop(0, IST, step=2)
    def _(k):
        for b in range(2):
            c = k + b

            @pl.when(c + 1 < IST)
            def _():
                @pl.when(c >= 1)
                def _():
                    wait_write(1 - b)

                issue_gather(c + 1, 1 - b)

            wait_gather(b)
            pltpu.async_copy(bufs[b], out.at[pl.ds(sid * EHG + c * GC, GC)],
                             semw[b])

    wait_write(0)
    wait_write(1)


def _make_gather_body(half):
    def body(row2, col2, tsrc, tdst, gsrc, gdst, idx, buf0, buf1, tab,
             semg0, semg1, semw0, semw1):
        cid = lax.axis_index("c")
        sid = lax.axis_index("s")

        @pl.when(cid == 0)
        def _():
            pltpu.sync_copy(tsrc.at[pl.ds(sid * NTS, NTS)],
                            tab.at[pl.ds(sid * NTS, NTS)])

        @pl.when(cid == 1)
        def _():
            pltpu.sync_copy(tdst.at[pl.ds(sid * NTS, NTS)],
                            tab.at[pl.ds(sid * NTS, NTS)])

        plsc.subcore_barrier()

        @pl.when(cid == 0)
        def _():
            _gather_pipe(half, row2, gsrc, sid, idx, buf0, buf1, tab,
                         semg0, semg1, semw0, semw1)

        @pl.when(cid == 1)
        def _():
            _gather_pipe(half, col2, gdst, sid, idx, buf0, buf1, tab,
                         semg0, semg1, semw0, semw1)

    return body


@functools.cache
def _gather_call(half):
    return pl.kernel(
        _make_gather_body(half),
        out_type=(jax.ShapeDtypeStruct((EH, TW), F32),
                  jax.ShapeDtypeStruct((EH, TW), F32)),
        mesh=_sc_mesh(),
        scratch_types=[
            pltpu.VMEM((IST, GC), jnp.int32),
            pltpu.VMEM((GC, TW), F32),
            pltpu.VMEM((GC, TW), F32),
            pltpu.VMEM_SHARED((NT, TW), F32),
            pltpu.SemaphoreType.DMA,
            pltpu.SemaphoreType.DMA,
            pltpu.SemaphoreType.DMA,
            pltpu.SemaphoreType.DMA,
        ],
    )


def _gather_pallas(half, row2, col2, tsrc, tdst):
    return _gather_call(half)(row2, col2, tsrc, tdst)


EHS = EH // 32
NCHH = EHS // GC


def _make_scatter_body(half):
    def body(row2, pay, zeros, part, idx2, buf0, buf1, acc, seml0, seml1,
             sems0, sems1):
        cid = lax.axis_index("c")
        sid = lax.axis_index("s")
        wid = sid * 2 + cid
        bufs = (buf0, buf1)
        seml = (seml0, seml1)
        sems = (sems0, sems1)

        @pl.when(sid == 0)
        def _():
            pltpu.sync_copy(zeros, acc)

        pltpu.sync_copy(
            row2.at[pl.ds(half * (EH // GC) + wid * NCHH, NCHH)], idx2)
        plsc.subcore_barrier()

        def issue_load(c, b):
            pltpu.async_copy(pay.at[pl.ds(wid * EHS + c * GC, GC)], bufs[b],
                             seml[b])

        def wait_load(b):
            pltpu.make_async_copy(pay.at[pl.ds(0, GC)], bufs[b],
                                  seml[b]).wait()

        def wait_scat(b):
            pltpu.make_async_copy(bufs[b], acc.at[idx2.at[0]],
                                  sems[b]).wait()

        issue_load(0, 0)

        @pl.loop(0, NCHH, step=2)
        def _(i):
            for b in range(2):
                c = i + b
                wait_load(b)
                pltpu.async_copy(bufs[b], acc.at[idx2.at[c]], sems[b],
                                 add=True)

                @pl.when(c + 1 < NCHH)
                def _():
                    @pl.when(c >= 1)
                    def _():
                        wait_scat(1 - b)

                    issue_load(c + 1, 1 - b)

        wait_scat(0)
        wait_scat(1)
        plsc.subcore_barrier()

        @pl.when(sid == 0)
        def _():
            pltpu.sync_copy(acc, part.at[cid])

    return body


@functools.cache
def _scatter_call(half):
    return pl.kernel(
        _make_scatter_body(half),
        out_type=jax.ShapeDtypeStruct((2, NT, TW), F32),
        mesh=_sc_mesh(),
        scratch_types=[
            pltpu.VMEM((NCHH, GC), jnp.int32),
            pltpu.VMEM((GC, TW), F32),
            pltpu.VMEM((GC, TW), F32),
            pltpu.VMEM_SHARED((NT, TW), F32),
            pltpu.SemaphoreType.DMA,
            pltpu.SemaphoreType.DMA,
            pltpu.SemaphoreType.DMA,
            pltpu.SemaphoreType.DMA,
        ],
    )


def _scatter_pallas(half, row2, payload, zeros):
    return _scatter_call(half)(row2, payload, zeros)


def _tables(hs, hd, x):
    z = jnp.zeros((hs.shape[0], TW - H - 3), F32)
    return (jnp.concatenate([hs, x, z], axis=1),
            jnp.concatenate([hd, x, z], axis=1))


def _node0_body(his_ref, wemb_ref, bemb_ref, loc_ref, vel_ref, wsrc_ref,
                wdst_ref, wv1_ref, bv1_ref, wv2t_ref, bv2_ref,
                h_ref, ts_ref, td_ref, vmod_ref, nvel_ref):
    h = jnp.dot(his_ref[...], wemb_ref[...], preferred_element_type=F32)
    h = h + bemb_ref[...]
    h_ref[...] = h
    hs = jnp.dot(h, wsrc_ref[...], preferred_element_type=F32)
    hd = jnp.dot(h, wdst_ref[...], preferred_element_type=F32)
    ts_ref[...], td_ref[...] = _tables(hs, hd, loc_ref[...])
    q = _silu(jnp.dot(h, wv1_ref[...], preferred_element_type=F32)
              + bv1_ref[...])
    vmod_ref[...] = (jnp.sum(q * wv2t_ref[...], axis=1, keepdims=True)
                     + bv2_ref[...])
    v = vel_ref[...]
    nrm = jnp.sqrt(jnp.sum(v * v, axis=1, keepdims=True)) + 1e-8
    nvel_ref[...] = v / nrm


NB = 1024
NNB = NT // NB


def _node0_pallas(his, loc, vel, wemb, bemb, wsrc, wdst, wv1, bv1, wv2t, bv2):
    blk = lambda c: pl.BlockSpec((NB, c), lambda i: (i, 0))
    rep = lambda r, c: pl.BlockSpec((r, c), lambda i: (0, 0))
    return pl.pallas_call(
        _node0_body,
        grid=(NNB,),
        in_specs=[
            blk(128), rep(128, H), rep(1, H), blk(3), blk(3), rep(H, H),
            rep(H, H), rep(H, H), rep(1, H), rep(1, H), rep(1, 1),
        ],
        out_specs=(blk(H), blk(TW), blk(TW), blk(1), blk(3)),
        out_shape=(
            jax.ShapeDtypeStruct((NT, H), F32),
            jax.ShapeDtypeStruct((NT, TW), F32),
            jax.ShapeDtypeStruct((NT, TW), F32),
            jax.ShapeDtypeStruct((NT, 1), F32),
            jax.ShapeDtypeStruct((NT, 3), F32),
        ),
    )(his, wemb, bemb, loc, vel, wsrc, wdst, wv1, bv1, wv2t, bv2)


def _eaproj_body(ea_ref, wea_ref, be1_ref, out_ref):
    a = ea_ref[...]
    wea = wea_ref[...]
    be1 = be1_ref[...]
    pieces = [
        jnp.dot(a[:, 16 * j:16 * (j + 1)], wea, preferred_element_type=F32)
        + be1
        for j in range(8)
    ]
    out_ref[...] = jnp.stack(pieces, axis=1).reshape(EB, H).astype(BF16)


def _eaproj_pallas(ea8, wea, be1):
    return pl.pallas_call(
        _eaproj_body,
        grid=(NEB,),
        in_specs=[
            pl.BlockSpec((EB // 8, 128), lambda i: (i, 0)),
            pl.BlockSpec((D_EDGE, H), lambda i: (0, 0)),
            pl.BlockSpec((1, H), lambda i: (0, 0)),
        ],
        out_specs=pl.BlockSpec((EB, H), lambda i: (i, 0)),
        out_shape=jax.ShapeDtypeStruct((EPAD, H), BF16),
    )(ea8, wea, be1)


def _edge_body(gs_ref, gd_ref, eab_ref, radw_ref, we2_ref, be2_ref,
               wc1_ref, bc1_ref, wc2t_ref, out_ref):
    gs = gs_ref[...].astype(F32)
    gd = gd_ref[...].astype(F32)
    cd = gs[:, H:H + 16] - gd[:, H:H + 16]
    radial = jnp.sum(cd * cd, axis=1, keepdims=True)
    pre1 = (gs[:, 0:H] + gd[:, 0:H] + eab_ref[...].astype(F32)
            + radial * radw_ref[...])
    m1 = _silu(pre1)
    m = _silu(jnp.dot(m1, we2_ref[...], preferred_element_type=F32)
              + be2_ref[...])
    q = _silu(jnp.dot(m, wc1_ref[...], preferred_element_type=F32)
              + bc1_ref[...])
    cm = jnp.sum(q * wc2t_ref[...], axis=1, keepdims=True)
    t16 = cd * cm
    lane = lax.broadcasted_iota(jnp.int32, t16.shape, 1)
    t16 = jnp.where(lane == 3, 1.0, t16)
    out_ref[...] = jnp.concatenate(
        [t16, m, jnp.zeros((t16.shape[0], TW - 16 - H), F32)], axis=1)


def _edge_pallas(half, gsrc, gdst, eab, radw, we2, be2, wc1, bc1, wc2t):
    full = lambda s: pl.BlockSpec(s, lambda i: tuple(0 for _ in s))
    off = half * (EH // EB)
    return pl.pallas_call(
        _edge_body,
        grid=(EH // EB,),
        in_specs=[
            pl.BlockSpec((EB, TW), lambda i: (i, 0)),
            pl.BlockSpec((EB, TW), lambda i: (i, 0)),
            pl.BlockSpec((EB, H), lambda i: (i + off, 0)),
            full((1, H)),
            full((H, H)),
            full((1, H)),
            full((H, H)),
            full((1, H)),
            full((1, H)),
        ],
        out_specs=pl.BlockSpec((EB, TW), lambda i: (i, 0)),
        out_shape=jax.ShapeDtypeStruct((EH, TW), F32),
    )(gsrc, gdst, eab, radw, we2, be2, wc1, bc1, wc2t)


def _node_body(p_ref, q_ref, h_ref, x_ref, v_ref, nvel_ref, vmod_ref,
               wnh_ref, wnm_ref, bn1_ref, wn2_ref, bn2_ref,
               wsrc_ref, wdst_ref, wv1_ref, bv1_ref, wv2t_ref, bv2_ref,
               hn_ref, xn_ref, vn_ref, ts_ref, td_ref, vmodn_ref):
    ps = (p_ref[0] + p_ref[1]) + (q_ref[0] + q_ref[1])
    cnt = jnp.maximum(ps[:, 3:4], 1.0)
    agg = ps[:, 0:3] / cnt
    v = v_ref[...] + agg + vmod_ref[...] * nvel_ref[...]
    vn_ref[...] = v
    xn = x_ref[...] + v
    xn_ref[...] = xn
    magg = ps[:, 16:80]
    h = h_ref[...]
    hmid = _silu(jnp.dot(h, wnh_ref[...], preferred_element_type=F32)
                 + jnp.dot(magg, wnm_ref[...], preferred_element_type=F32)
                 + bn1_ref[...])
    hn = h + jnp.dot(hmid, wn2_ref[...], preferred_element_type=F32) \
        + bn2_ref[...]
    hn_ref[...] = hn
    hs = jnp.dot(hn, wsrc_ref[...], preferred_element_type=F32)
    hd = jnp.dot(hn, wdst_ref[...], preferred_element_type=F32)
    ts_ref[...], td_ref[...] = _tables(hs, hd, xn)
    q = _silu(jnp.dot(hn, wv1_ref[...], preferred_element_type=F32)
              + bv1_ref[...])
    vmodn_ref[...] = (jnp.sum(q * wv2t_ref[...], axis=1, keepdims=True)
                      + bv2_ref[...])


def _node_pallas(pa, pb, h, x, v, nvel, vmod, wnh, wnm, bn1, wn2, bn2,
                 wsrc, wdst, wv1, bv1, wv2t, bv2):
    blk = lambda c: pl.BlockSpec((NB, c), lambda i: (i, 0))
    rep = lambda r, c: pl.BlockSpec((r, c), lambda i: (0, 0))
    pspec = pl.BlockSpec((2, NB, TW), lambda i: (0, i, 0))
    return pl.pallas_call(
        _node_body,
        grid=(NNB,),
        in_specs=[
            pspec, pspec,
            blk(H), blk(3), blk(3), blk(3), blk(1),
            rep(H, H), rep(H, H), rep(1, H), rep(H, H), rep(1, H),
            rep(H, H), rep(H, H), rep(H, H), rep(1, H), rep(1, H), rep(1, 1),
        ],
        out_specs=(blk(H), blk(3), blk(3), blk(TW), blk(TW), blk(1)),
        out_shape=(
            jax.ShapeDtypeStruct((NT, H), F32),
            jax.ShapeDtypeStruct((NT, 3), F32),
            jax.ShapeDtypeStruct((NT, 3), F32),
            jax.ShapeDtypeStruct((NT, TW), F32),
            jax.ShapeDtypeStruct((NT, TW), F32),
            jax.ShapeDtypeStruct((NT, 1), F32),
        ),
    )(pa, pb, h, x, v, nvel, vmod, wnh, wnm, bn1, wn2, bn2,
      wsrc, wdst, wv1, bv1, wv2t, bv2)


def kernel(his, loc, edges, vel, edge_attr, W_emb, b_emb, We1, be1, We2, be2,
           Wn1, bn1, Wn2, bn2, Wc1, bc1, Wc2, Wv1, bv1, Wv2, bv2):
    row = edges[0]
    col = edges[1]
    pad = jnp.full((EPAD - E,), N, jnp.int32)
    row2 = jnp.concatenate([row, pad]).reshape(EPAD // GC, GC)
    col2 = jnp.concatenate([col, pad]).reshape(EPAD // GC, GC)
    ea8 = jnp.pad(edge_attr.reshape(E // 8, 128),
                  ((0, (EPAD - E) // 8), (0, 0)))
    zeros = jnp.zeros((NT, TW), F32)

    wsrc = We1[0:H]
    wdst = We1[H:2 * H]
    radw = We1[2 * H:2 * H + 1]
    wea = We1[2 * H + 1:]
    wnh = Wn1[0:H]
    wnm = Wn1[H:2 * H]
    b_emb2 = b_emb.reshape(1, H)
    be1_2 = be1.reshape(1, H)
    be2_2 = be2.reshape(1, H)
    bn1_2 = bn1.reshape(1, H)
    bn2_2 = bn2.reshape(1, H)
    bc1_2 = bc1.reshape(1, H)
    bv1_2 = bv1.reshape(1, H)
    wc2t = Wc2.reshape(1, H)
    wv2t = Wv2.reshape(1, H)
    bv2_2 = bv2.reshape(1, 1)

    his_p = jnp.pad(his, ((0, NT - N), (0, 0)))
    loc_p = jnp.pad(loc, ((0, NT - N), (0, 0)))
    vel_p = jnp.pad(vel, ((0, NT - N), (0, 0)))

    h, tsrc, tdst, vmod, nvel = _node0_pallas(
        his_p, loc_p, vel_p, W_emb, b_emb2, wsrc, wdst, Wv1, bv1_2, wv2t,
        bv2_2)
    eab = _eaproj_pallas(ea8, wea, be1_2)

    x = loc_p
    v = vel_p
    for _ in range(4):
        ga = _gather_pallas(0, row2, col2, tsrc, tdst)
        gb = _gather_pallas(1, row2, col2, tsrc, tdst)
        paya = _edge_pallas(0, ga[0], ga[1], eab, radw, We2, be2_2,
                            Wc1, bc1_2, wc2t)
        payb = _edge_pallas(1, gb[0], gb[1], eab, radw, We2, be2_2,
                            Wc1, bc1_2, wc2t)
        pa = _scatter_pallas(0, row2, paya, zeros)
        pb = _scatter_pallas(1, row2, payb, zeros)
        h, x, v, tsrc, tdst, vmod = _node_pallas(
            pa, pb, h, x, v, nvel, vmod, wnh, wnm, bn1_2, Wn2, bn2_2,
            wsrc, wdst, Wv1, bv1_2, wv2t, bv2_2)
    return x[:N], h[:N], v[:N]


# speedup vs baseline: 2.2778x; 1.0007x over previous
import functools

import jax
import jax.numpy as jnp
from jax import lax
from jax.experimental import pallas as pl
from jax.experimental.pallas import tpu as pltpu
from jax.experimental.pallas import tpu_sc as plsc

N = 10000
E = 320000
H = 64
D_EDGE = 16

NT = 10240
TW = 128
EPAD = 327680
GC = 128
EB = 2048
NEB = EPAD // EB

F32 = jnp.float32
BF16 = jnp.bfloat16

@functools.cache
def _sc_mesh():
    return plsc.VectorSubcoreMesh(core_axis_name="c", subcore_axis_name="s",
                                  num_cores=2, num_subcores=16)


def _silu(x):
    return x * jax.nn.sigmoid(x)


EH = EPAD // 2
EHG = EH // 16
IST = EHG // GC
NTS = NT // 16


def _gather_pipe(half, idx2, out, sid, idx, buf0, buf1, tab, semg0, semg1,
                 semw0, semw1):
    bufs = (buf0, buf1)
    semg = (semg0, semg1)
    semw = (semw0, semw1)

    def issue_gather(c, b):
        pltpu.async_copy(tab.at[idx.at[c]], bufs[b], semg[b])

    def wait_gather(b):
        pltpu.make_async_copy(tab.at[idx.at[0]], bufs[b], semg[b]).wait()

    def wait_write(b):
        pltpu.make_async_copy(bufs[b], out.at[pl.ds(0, GC)], semw[b]).wait()

    pltpu.sync_copy(
        idx2.at[pl.ds(half * (EH // GC) + sid * IST, IST)], idx)
    issue_gather(0, 0)

    @pl.loop(0, IST, step=2)
    def _(k):
        for b in range(2):
            c = k + b

            @pl.when(c + 1 < IST)
            def _():
                @pl.when(c >= 1)
                def _():
                    wait_write(1 - b)

                issue_gather(c + 1, 1 - b)

            wait_gather(b)
            pltpu.async_copy(bufs[b], out.at[pl.ds(sid * EHG + c * GC, GC)],
                             semw[b])

    wait_write(0)
    wait_write(1)


def _make_gather_body(half):
    def body(row2, col2, tsrc, tdst, gsrc, gdst, idx, buf0, buf1, tab,
             semg0, semg1, semw0, semw1):
        cid = lax.axis_index("c")
        sid = lax.axis_index("s")

        @pl.when(cid == 0)
        def _():
            pltpu.sync_copy(tsrc.at[pl.ds(sid * NTS, NTS)],
                            tab.at[pl.ds(sid * NTS, NTS)])

        @pl.when(cid == 1)
        def _():
            pltpu.sync_copy(tdst.at[pl.ds(sid * NTS, NTS)],
                            tab.at[pl.ds(sid * NTS, NTS)])

        plsc.subcore_barrier()

        @pl.when(cid == 0)
        def _():
            _gather_pipe(half, row2, gsrc, sid, idx, buf0, buf1, tab,
                         semg0, semg1, semw0, semw1)

        @pl.when(cid == 1)
        def _():
            _gather_pipe(half, col2, gdst, sid, idx, buf0, buf1, tab,
                         semg0, semg1, semw0, semw1)

    return body


@functools.cache
def _gather_call(half):
    return pl.kernel(
        _make_gather_body(half),
        out_type=(jax.ShapeDtypeStruct((EH, TW), F32),
                  jax.ShapeDtypeStruct((EH, TW), F32)),
        mesh=_sc_mesh(),
        scratch_types=[
            pltpu.VMEM((IST, GC), jnp.int32),
            pltpu.VMEM((GC, TW), F32),
            pltpu.VMEM((GC, TW), F32),
            pltpu.VMEM_SHARED((NT, TW), F32),
            pltpu.SemaphoreType.DMA,
            pltpu.SemaphoreType.DMA,
            pltpu.SemaphoreType.DMA,
            pltpu.SemaphoreType.DMA,
        ],
    )


def _gather_pallas(half, row2, col2, tsrc, tdst):
    return _gather_call(half)(row2, col2, tsrc, tdst)


EHS = EH // 32
NCHH = EHS // GC


def _make_scatter_body(half):
    def body(row2, pay, zeros, part, idx2, buf0, buf1, acc, seml0, seml1,
             sems0, sems1):
        cid = lax.axis_index("c")
        sid = lax.axis_index("s")
        wid = sid * 2 + cid
        bufs = (buf0, buf1)
        seml = (seml0, seml1)
        sems = (sems0, sems1)

        pltpu.sync_copy(zeros.at[pl.ds(sid * NTS, NTS)],
                        acc.at[pl.ds(sid * NTS, NTS)])
        pltpu.sync_copy(
            row2.at[pl.ds(half * (EH // GC) + wid * NCHH, NCHH)], idx2)
        plsc.subcore_barrier()

        def issue_load(c, b):
            pltpu.async_copy(pay.at[pl.ds(wid * EHS + c * GC, GC)], bufs[b],
                             seml[b])

        def wait_load(b):
            pltpu.make_async_copy(pay.at[pl.ds(0, GC)], bufs[b],
                                  seml[b]).wait()

        def wait_scat(b):
            pltpu.make_async_copy(bufs[b], acc.at[idx2.at[0]],
                                  sems[b]).wait()

        issue_load(0, 0)

        @pl.loop(0, NCHH, step=2)
        def _(i):
            for b in range(2):
                c = i + b
                wait_load(b)
                pltpu.async_copy(bufs[b], acc.at[idx2.at[c]], sems[b],
                                 add=True)

                @pl.when(c + 1 < NCHH)
                def _():
                    @pl.when(c >= 1)
                    def _():
                        wait_scat(1 - b)

                    issue_load(c + 1, 1 - b)

        wait_scat(0)
        wait_scat(1)
        plsc.subcore_barrier()
        pltpu.sync_copy(acc.at[pl.ds(sid * NTS, NTS)],
                        part.at[cid, pl.ds(sid * NTS, NTS)])

    return body


@functools.cache
def _scatter_call(half):
    return pl.kernel(
        _make_scatter_body(half),
        out_type=jax.ShapeDtypeStruct((2, NT, TW), F32),
        mesh=_sc_mesh(),
        scratch_types=[
            pltpu.VMEM((NCHH, GC), jnp.int32),
            pltpu.VMEM((GC, TW), F32),
            pltpu.VMEM((GC, TW), F32),
            pltpu.VMEM_SHARED((NT, TW), F32),
            pltpu.SemaphoreType.DMA,
            pltpu.SemaphoreType.DMA,
            pltpu.SemaphoreType.DMA,
            pltpu.SemaphoreType.DMA,
        ],
    )


def _scatter_pallas(half, row2, payload, zeros):
    return _scatter_call(half)(row2, payload, zeros)


def _tables(hs, hd, x):
    z = jnp.zeros((hs.shape[0], TW - H - 3), F32)
    return (jnp.concatenate([hs, x, z], axis=1),
            jnp.concatenate([hd, x, z], axis=1))


def _node0_body(his_ref, wemb_ref, bemb_ref, loc_ref, vel_ref, wsrc_ref,
                wdst_ref, wv1_ref, bv1_ref, wv2t_ref, bv2_ref,
                h_ref, ts_ref, td_ref, vmod_ref, nvel_ref):
    h = jnp.dot(his_ref[...], wemb_ref[...], preferred_element_type=F32)
    h = h + bemb_ref[...]
    h_ref[...] = h
    hs = jnp.dot(h, wsrc_ref[...], preferred_element_type=F32)
    hd = jnp.dot(h, wdst_ref[...], preferred_element_type=F32)
    ts_ref[...], td_ref[...] = _tables(hs, hd, loc_ref[...])
    q = _silu(jnp.dot(h, wv1_ref[...], preferred_element_type=F32)
              + bv1_ref[...])
    vmod_ref[...] = (jnp.sum(q * wv2t_ref[...], axis=1, keepdims=True)
                     + bv2_ref[...])
    v = vel_ref[...]
    nrm = jnp.sqrt(jnp.sum(v * v, axis=1, keepdims=True)) + 1e-8
    nvel_ref[...] = v / nrm


NB = 1024
NNB = NT // NB


def _node0_pallas(his, loc, vel, wemb, bemb, wsrc, wdst, wv1, bv1, wv2t, bv2):
    blk = lambda c: pl.BlockSpec((NB, c), lambda i: (i, 0))
    rep = lambda r, c: pl.BlockSpec((r, c), lambda i: (0, 0))
    return pl.pallas_call(
        _node0_body,
        grid=(NNB,),
        in_specs=[
            blk(128), rep(128, H), rep(1, H), blk(3), blk(3), rep(H, H),
            rep(H, H), rep(H, H), rep(1, H), rep(1, H), rep(1, 1),
        ],
        out_specs=(blk(H), blk(TW), blk(TW), blk(1), blk(3)),
        out_shape=(
            jax.ShapeDtypeStruct((NT, H), F32),
            jax.ShapeDtypeStruct((NT, TW), F32),
            jax.ShapeDtypeStruct((NT, TW), F32),
            jax.ShapeDtypeStruct((NT, 1), F32),
            jax.ShapeDtypeStruct((NT, 3), F32),
        ),
    )(his, wemb, bemb, loc, vel, wsrc, wdst, wv1, bv1, wv2t, bv2)


def _eaproj_body(ea_ref, wea_ref, be1_ref, out_ref):
    a = ea_ref[...]
    wea = wea_ref[...]
    be1 = be1_ref[...]
    pieces = [
        jnp.dot(a[:, 16 * j:16 * (j + 1)], wea, preferred_element_type=F32)
        + be1
        for j in range(8)
    ]
    out_ref[...] = jnp.stack(pieces, axis=1).reshape(EB, H).astype(BF16)


def _eaproj_pallas(ea8, wea, be1):
    return pl.pallas_call(
        _eaproj_body,
        grid=(NEB,),
        in_specs=[
            pl.BlockSpec((EB // 8, 128), lambda i: (i, 0)),
            pl.BlockSpec((D_EDGE, H), lambda i: (0, 0)),
            pl.BlockSpec((1, H), lambda i: (0, 0)),
        ],
        out_specs=pl.BlockSpec((EB, H), lambda i: (i, 0)),
        out_shape=jax.ShapeDtypeStruct((EPAD, H), BF16),
    )(ea8, wea, be1)


def _edge_body(gs_ref, gd_ref, eab_ref, radw_ref, we2_ref, be2_ref,
               wc1_ref, bc1_ref, wc2t_ref, out_ref):
    gs = gs_ref[...].astype(F32)
    gd = gd_ref[...].astype(F32)
    cd = gs[:, H:H + 16] - gd[:, H:H + 16]
    radial = jnp.sum(cd * cd, axis=1, keepdims=True)
    pre1 = (gs[:, 0:H] + gd[:, 0:H] + eab_ref[...].astype(F32)
            + radial * radw_ref[...])
    m1 = _silu(pre1)
    m = _silu(jnp.dot(m1, we2_ref[...], preferred_element_type=F32)
              + be2_ref[...])
    q = _silu(jnp.dot(m, wc1_ref[...], preferred_element_type=F32)
              + bc1_ref[...])
    cm = jnp.sum(q * wc2t_ref[...], axis=1, keepdims=True)
    t16 = cd * cm
    lane = lax.broadcasted_iota(jnp.int32, t16.shape, 1)
    t16 = jnp.where(lane == 3, 1.0, t16)
    out_ref[...] = jnp.concatenate(
        [t16, m, jnp.zeros((t16.shape[0], TW - 16 - H), F32)], axis=1)


def _edge_pallas(half, gsrc, gdst, eab, radw, we2, be2, wc1, bc1, wc2t):
    full = lambda s: pl.BlockSpec(s, lambda i: tuple(0 for _ in s))
    off = half * (EH // EB)
    return pl.pallas_call(
        _edge_body,
        grid=(EH // EB,),
        in_specs=[
            pl.BlockSpec((EB, TW), lambda i: (i, 0)),
            pl.BlockSpec((EB, TW), lambda i: (i, 0)),
            pl.BlockSpec((EB, H), lambda i: (i + off, 0)),
            full((1, H)),
            full((H, H)),
            full((1, H)),
            full((H, H)),
            full((1, H)),
            full((1, H)),
        ],
        out_specs=pl.BlockSpec((EB, TW), lambda i: (i, 0)),
        out_shape=jax.ShapeDtypeStruct((EH, TW), F32),
    )(gsrc, gdst, eab, radw, we2, be2, wc1, bc1, wc2t)


def _node_body(p_ref, q_ref, h_ref, x_ref, v_ref, nvel_ref, vmod_ref,
               wnh_ref, wnm_ref, bn1_ref, wn2_ref, bn2_ref,
               wsrc_ref, wdst_ref, wv1_ref, bv1_ref, wv2t_ref, bv2_ref,
               hn_ref, xn_ref, vn_ref, ts_ref, td_ref, vmodn_ref):
    ps = (p_ref[0] + p_ref[1]) + (q_ref[0] + q_ref[1])
    cnt = jnp.maximum(ps[:, 3:4], 1.0)
    agg = ps[:, 0:3] / cnt
    v = v_ref[...] + agg + vmod_ref[...] * nvel_ref[...]
    vn_ref[...] = v
    xn = x_ref[...] + v
    xn_ref[...] = xn
    magg = ps[:, 16:80]
    h = h_ref[...]
    hmid = _silu(jnp.dot(h, wnh_ref[...], preferred_element_type=F32)
                 + jnp.dot(magg, wnm_ref[...], preferred_element_type=F32)
                 + bn1_ref[...])
    hn = h + jnp.dot(hmid, wn2_ref[...], preferred_element_type=F32) \
        + bn2_ref[...]
    hn_ref[...] = hn
    hs = jnp.dot(hn, wsrc_ref[...], preferred_element_type=F32)
    hd = jnp.dot(hn, wdst_ref[...], preferred_element_type=F32)
    ts_ref[...], td_ref[...] = _tables(hs, hd, xn)
    q = _silu(jnp.dot(hn, wv1_ref[...], preferred_element_type=F32)
              + bv1_ref[...])
    vmodn_ref[...] = (jnp.sum(q * wv2t_ref[...], axis=1, keepdims=True)
                      + bv2_ref[...])


def _node_pallas(pa, pb, h, x, v, nvel, vmod, wnh, wnm, bn1, wn2, bn2,
                 wsrc, wdst, wv1, bv1, wv2t, bv2):
    blk = lambda c: pl.BlockSpec((NB, c), lambda i: (i, 0))
    rep = lambda r, c: pl.BlockSpec((r, c), lambda i: (0, 0))
    pspec = pl.BlockSpec((2, NB, TW), lambda i: (0, i, 0))
    return pl.pallas_call(
        _node_body,
        grid=(NNB,),
        in_specs=[
            pspec, pspec,
            blk(H), blk(3), blk(3), blk(3), blk(1),
            rep(H, H), rep(H, H), rep(1, H), rep(H, H), rep(1, H),
            rep(H, H), rep(H, H), rep(H, H), rep(1, H), rep(1, H), rep(1, 1),
        ],
        out_specs=(blk(H), blk(3), blk(3), blk(TW), blk(TW), blk(1)),
        out_shape=(
            jax.ShapeDtypeStruct((NT, H), F32),
            jax.ShapeDtypeStruct((NT, 3), F32),
            jax.ShapeDtypeStruct((NT, 3), F32),
            jax.ShapeDtypeStruct((NT, TW), F32),
            jax.ShapeDtypeStruct((NT, TW), F32),
            jax.ShapeDtypeStruct((NT, 1), F32),
        ),
    )(pa, pb, h, x, v, nvel, vmod, wnh, wnm, bn1, wn2, bn2,
      wsrc, wdst, wv1, bv1, wv2t, bv2)


def kernel(his, loc, edges, vel, edge_attr, W_emb, b_emb, We1, be1, We2, be2,
           Wn1, bn1, Wn2, bn2, Wc1, bc1, Wc2, Wv1, bv1, Wv2, bv2):
    row = edges[0]
    col = edges[1]
    pad = jnp.full((EPAD - E,), N, jnp.int32)
    row2 = jnp.concatenate([row, pad]).reshape(EPAD // GC, GC)
    col2 = jnp.concatenate([col, pad]).reshape(EPAD // GC, GC)
    ea8 = jnp.pad(edge_attr.reshape(E // 8, 128),
                  ((0, (EPAD - E) // 8), (0, 0)))
    zeros = jnp.zeros((NT, TW), F32)

    wsrc = We1[0:H]
    wdst = We1[H:2 * H]
    radw = We1[2 * H:2 * H + 1]
    wea = We1[2 * H + 1:]
    wnh = Wn1[0:H]
    wnm = Wn1[H:2 * H]
    b_emb2 = b_emb.reshape(1, H)
    be1_2 = be1.reshape(1, H)
    be2_2 = be2.reshape(1, H)
    bn1_2 = bn1.reshape(1, H)
    bn2_2 = bn2.reshape(1, H)
    bc1_2 = bc1.reshape(1, H)
    bv1_2 = bv1.reshape(1, H)
    wc2t = Wc2.reshape(1, H)
    wv2t = Wv2.reshape(1, H)
    bv2_2 = bv2.reshape(1, 1)

    his_p = jnp.pad(his, ((0, NT - N), (0, 0)))
    loc_p = jnp.pad(loc, ((0, NT - N), (0, 0)))
    vel_p = jnp.pad(vel, ((0, NT - N), (0, 0)))

    h, tsrc, tdst, vmod, nvel = _node0_pallas(
        his_p, loc_p, vel_p, W_emb, b_emb2, wsrc, wdst, Wv1, bv1_2, wv2t,
        bv2_2)
    eab = _eaproj_pallas(ea8, wea, be1_2)

    x = loc_p
    v = vel_p
    for _ in range(4):
        ga = _gather_pallas(0, row2, col2, tsrc, tdst)
        gb = _gather_pallas(1, row2, col2, tsrc, tdst)
        paya = _edge_pallas(0, ga[0], ga[1], eab, radw, We2, be2_2,
                            Wc1, bc1_2, wc2t)
        payb = _edge_pallas(1, gb[0], gb[1], eab, radw, We2, be2_2,
                            Wc1, bc1_2, wc2t)
        pa = _scatter_pallas(0, row2, paya, zeros)
        pb = _scatter_pallas(1, row2, payb, zeros)
        h, x, v, tsrc, tdst, vmod = _node_pallas(
            pa, pb, h, x, v, nvel, vmod, wnh, wnm, bn1_2, Wn2, bn2_2,
            wsrc, wdst, Wv1, bv1_2, wv2t, bv2_2)
    return x[:N], h[:N], v[:N]
